# baseline (speedup 1.0000x reference)
import functools

import jax
import jax.numpy as jnp
from jax import lax
from jax.experimental import pallas as pl
from jax.experimental.pallas import tpu as pltpu
from jax.experimental.pallas import tpu_sc as plsc

T = 4096
HIDDEN = 1024
NH = 16
HS = 64
CHUNK = 64
NHASH = 2
NB = 128
HASH_SEED = 1234
MASKVAL = -1e5
S = NHASH * T
NCH = S // CHUNK
R = NH * S


def _k1_body(x_ref, w_ref, o_ref):
    o_ref[...] = jnp.dot(x_ref[...], w_ref[0],
                         preferred_element_type=jnp.float32)


def _project(x, w_heads):
    return pl.pallas_call(
        _k1_body,
        grid=(4, NH),
        in_specs=[
            pl.BlockSpec((1024, HIDDEN), lambda i, h: (i, 0)),
            pl.BlockSpec((1, HIDDEN, HS), lambda i, h: (h, 0, 0)),
        ],
        out_specs=pl.BlockSpec((1024, HS), lambda i, h: (h * 4 + i, 0)),
        out_shape=jax.ShapeDtypeStruct((NH * T, HS), jnp.float32),
    )(x, w_heads)


def _k2_body(x_ref, rot_ref, rank_ref):
    x = x_ref[...]
    rot = rot_ref[0, 0]
    r = jnp.dot(x, rot, preferred_element_type=jnp.float32)

    ii = lax.broadcasted_iota(jnp.int32, (T, HS), 1).astype(jnp.float32)
    mx = jnp.max(r, axis=1, keepdims=True)
    mn = jnp.min(r, axis=1, keepdims=True)
    a1 = jnp.min(jnp.where(r == mx, ii, 64.0), axis=1, keepdims=True)
    a2 = jnp.min(jnp.where(r == mn, ii, 64.0), axis=1, keepdims=True) + 64.0
    bucket = jnp.where(mx >= -mn, a1, a2)

    bi = lax.broadcasted_iota(jnp.int32, (128, NB), 1).astype(jnp.float32)
    rows = lax.broadcasted_iota(jnp.int32, (128, 128), 0)
    cols = lax.broadcasted_iota(jnp.int32, (128, 128), 1)
    lstrict = (rows > cols).astype(jnp.float32)
    ustrict = (rows < cols).astype(jnp.float32)

    run = jnp.zeros((1, NB), jnp.float32)
    bases = []
    for blk in range(32):
        ob = (bucket[blk * 128:(blk + 1) * 128] == bi).astype(jnp.float32)
        bases.append(run)
        run = run + jnp.sum(ob, axis=0, keepdims=True)
    binbase = jnp.dot(run, ustrict, preferred_element_type=jnp.float32)

    for blk in range(32):
        ob = (bucket[blk * 128:(blk + 1) * 128] == bi).astype(jnp.float32)
        pb = jnp.dot(lstrict, ob, preferred_element_type=jnp.float32)
        base = binbase + bases[blk]
        rk = jnp.sum(ob * base, axis=1, keepdims=True) + \
             jnp.sum(ob * pb, axis=1, keepdims=True)
        rank_ref[0, 0, blk * 128:(blk + 1) * 128, :] = rk.astype(jnp.int32)


def _buckets_ranks(tab, rot4):
    return pl.pallas_call(
        _k2_body,
        grid=(NH, NHASH),
        in_specs=[
            pl.BlockSpec((T, HS), lambda h, a: (h, 0)),
            pl.BlockSpec((1, 1, HS, HS), lambda h, a: (h, a, 0, 0)),
        ],
        out_specs=pl.BlockSpec((1, 1, T, 1), lambda h, a: (h, a, 0, 0)),
        out_shape=jax.ShapeDtypeStruct((NH, NHASH, T, 1), jnp.int32),
    )(tab, rot4)


_SC_MESH = plsc.VectorSubcoreMesh(core_axis_name="c", subcore_axis_name="s")
_SC_PARAMS = pltpu.CompilerParams(needs_layout_passes=False,
                                  use_tc_tiling_on_sc=False)
_Z16 = lambda: jnp.zeros((16,), jnp.int32)


def _sc_sort_gather(rank_q, rank_k, qtab, ktab, vtab):
    @functools.partial(
        pl.kernel,
        mesh=_SC_MESH,
        compiler_params=_SC_PARAMS,
        out_type=[
            jax.ShapeDtypeStruct((R, 1), jnp.float32),
            jax.ShapeDtypeStruct((NH, NCH, 1, CHUNK), jnp.float32),
            jax.ShapeDtypeStruct((R, HS), jnp.float32),
            jax.ShapeDtypeStruct((R, HS), jnp.float32),
            jax.ShapeDtypeStruct((R, HS), jnp.float32),
        ],
        scratch_types=[
            pltpu.VMEM((T, 1), jnp.int32),
            pltpu.VMEM((T,), jnp.int32),
            pltpu.VMEM((T,), jnp.int32),
            pltpu.VMEM((T, 1), jnp.float32),
            pltpu.VMEM((CHUNK, 1, CHUNK), jnp.float32),
            pltpu.VMEM((128, HS), jnp.float32),
            pltpu.SemaphoreType.DMA,
        ],
    )
    def k(rq, rk, qt, kt, vt, posq, poskl, qs, ks, vs,
          rank_vm, idxq_vm, idxk_vm, pcol_vm, plane_vm, rows_vm, sem):
        wid = lax.axis_index("s") * 2 + lax.axis_index("c")
        h = wid // 2
        a = wid % 2

        def load_rank(rref):
            pltpu.sync_copy(rref.at[h, a], rank_vm)

        def inv_q(j, _):
            rk16 = plsc.load_gather(rank_vm, [j * 16 + lax.iota(jnp.int32, 16),
                                              _Z16()])
            base = j * 16 + lax.iota(jnp.int32, 16)
            plsc.store_scatter(idxq_vm, [rk16], base + h * T)
            plsc.store_scatter(pcol_vm, [rk16, _Z16()],
                               base.astype(jnp.float32))
            return 0

        def inv_k(j, _):
            rk16 = plsc.load_gather(rank_vm, [j * 16 + lax.iota(jnp.int32, 16),
                                              _Z16()])
            base = j * 16 + lax.iota(jnp.int32, 16)
            plsc.store_scatter(idxk_vm, [rk16], base + h * T)
            plsc.store_scatter(
                plane_vm,
                [lax.shift_right_logical(rk16, 6), _Z16(),
                 lax.bitwise_and(rk16, 63)],
                base.astype(jnp.float32))
            return 0

        def gather(tab, idx_vm, dst):
            def g_step(j, _):
                src = tab.at[idx_vm.at[pl.ds(j * 128, 128)]]
                pltpu.async_copy(src, rows_vm, sem).wait()
                pltpu.sync_copy(rows_vm,
                                dst.at[pl.ds(wid * T + j * 128, 128)])
                return 0

            lax.fori_loop(0, T // 128, g_step, 0)

        load_rank(rq)
        lax.fori_loop(0, T // 16, inv_q, 0)
        pltpu.sync_copy(pcol_vm, posq.at[pl.ds(wid * T, T)])
        load_rank(rk)
        lax.fori_loop(0, T // 16, inv_k, 0)
        pltpu.sync_copy(plane_vm, poskl.at[h, pl.ds(a * CHUNK, CHUNK)])
        gather(qt, idxq_vm, qs)
        gather(kt, idxk_vm, ks)
        gather(vt, idxk_vm, vs)

    return k(rank_q, rank_k, qtab, ktab, vtab)


def _norm_k(rows):
    var = jnp.mean(rows * rows, axis=1, keepdims=True)
    return rows * lax.rsqrt(var + 1e-6) * (HS ** -0.5)


def _k4_body(qs_ref, ks_ref, vs_ref, pq_ref, pkl_ref,
             kw_ref, vw_ref, pkw_ref,
             out_ref, lg_ref, pk_scr, pv_scr, pp_scr):
    g = pl.program_id(1)

    @pl.when(g == 0)
    def _init():
        pk_scr[...] = _norm_k(kw_ref[...])
        pv_scr[...] = vw_ref[...]
        pp_scr[...] = pkw_ref[0, 0]

    kprev = pk_scr[...]
    vprev = pv_scr[...]
    pprev = pp_scr[...]
    for j in range(16):
        qc = qs_ref[j * 64:(j + 1) * 64, :]
        kn = _norm_k(ks_ref[j * 64:(j + 1) * 64, :])
        vc = vs_ref[j * 64:(j + 1) * 64, :]
        pkc = pkl_ref[0, j, :, :]
        pqc = pq_ref[j * 64:(j + 1) * 64, :]
        nt = (((1,), (1,)), ((), ()))
        d0 = lax.dot_general(qc, kprev, nt, preferred_element_type=jnp.float32)
        d1 = lax.dot_general(qc, kn, nt, preferred_element_type=jnp.float32)
        d0 = jnp.where(pqc != pprev, d0, MASKVAL)
        d1 = jnp.where(pqc != pkc, d1, MASKVAL)
        m = jnp.maximum(jnp.max(d0, axis=1, keepdims=True),
                        jnp.max(d1, axis=1, keepdims=True))
        e0 = jnp.exp(d0 - m)
        e1 = jnp.exp(d1 - m)
        s = jnp.sum(e0, axis=1, keepdims=True) + \
            jnp.sum(e1, axis=1, keepdims=True)
        o = jnp.dot(e0, vprev, preferred_element_type=jnp.float32) + \
            jnp.dot(e1, vc, preferred_element_type=jnp.float32)
        out_ref[j * 64:(j + 1) * 64, :] = o / s
        lg_ref[j * 64:(j + 1) * 64, :] = m + jnp.log(s)
        kprev, vprev, pprev = kn, vc, pkc
    pk_scr[...] = kprev
    pv_scr[...] = vprev
    pp_scr[...] = pprev


def _attention(qs, ks, vs, pq, pkl):
    G = S // 8
    return pl.pallas_call(
        _k4_body,
        grid=(NH, 8),
        in_specs=[
            pl.BlockSpec((G, HS), lambda h, g: (h * 8 + g, 0)),
            pl.BlockSpec((G, HS), lambda h, g: (h * 8 + g, 0)),
            pl.BlockSpec((G, HS), lambda h, g: (h * 8 + g, 0)),
            pl.BlockSpec((G, 1), lambda h, g: (h * 8 + g, 0)),
            pl.BlockSpec((1, 16, 1, HS), lambda h, g: (h, g, 0, 0)),
            pl.BlockSpec((64, HS), lambda h, g: (h * NCH + NCH - 1, 0)),
            pl.BlockSpec((64, HS), lambda h, g: (h * NCH + NCH - 1, 0)),
            pl.BlockSpec((1, 1, 1, HS), lambda h, g: (h, NCH - 1, 0, 0)),
        ],
        out_specs=[
            pl.BlockSpec((G, HS), lambda h, g: (h * 8 + g, 0)),
            pl.BlockSpec((G, 1), lambda h, g: (h * 8 + g, 0)),
        ],
        out_shape=[
            jax.ShapeDtypeStruct((R, HS), jnp.float32),
            jax.ShapeDtypeStruct((R, 1), jnp.float32),
        ],
        scratch_shapes=[
            pltpu.VMEM((64, HS), jnp.float32),
            pltpu.VMEM((64, HS), jnp.float32),
            pltpu.VMEM((1, HS), jnp.float32),
        ],
    )(qs, ks, vs, pq, pkl, ks, vs, pkl)


def _sc_unsort(rank_k, outs, lgs):
    @functools.partial(
        pl.kernel,
        mesh=_SC_MESH,
        compiler_params=_SC_PARAMS,
        out_type=[
            jax.ShapeDtypeStruct((R, HS), jnp.float32),
            jax.ShapeDtypeStruct((R, 1), jnp.float32),
        ],
        scratch_types=[
            pltpu.VMEM((T, 1), jnp.int32),
            pltpu.VMEM((T,), jnp.int32),
            pltpu.VMEM((T, 1), jnp.float32),
            pltpu.VMEM((T, 1), jnp.float32),
            pltpu.VMEM((128, HS), jnp.float32),
            pltpu.SemaphoreType.DMA,
        ],
    )
    def k(rk, osrc, lsrc, odst, ldst,
          rank_vm, idx_vm, lg_vm, lgo_vm, rows_vm, sem):
        wid = lax.axis_index("s") * 2 + lax.axis_index("c")
        h = wid // 2
        a = wid % 2
        base = wid * T
        pltpu.sync_copy(rk.at[h, a], rank_vm)
        pltpu.sync_copy(lsrc.at[pl.ds(base, T)], lg_vm)

        def lg_step(j, _):
            seq = j * 16 + lax.iota(jnp.int32, 16)
            r16 = plsc.load_gather(rank_vm, [seq, _Z16()])
            vals = plsc.load_gather(lg_vm, [r16, _Z16()])
            plsc.store_scatter(lgo_vm, [seq, _Z16()], vals)
            idx_vm[pl.ds(j * 16, 16)] = r16 + base
            return 0

        lax.fori_loop(0, T // 16, lg_step, 0)
        pltpu.sync_copy(lgo_vm, ldst.at[pl.ds(base, T)])

        def g_step(j, _):
            src = osrc.at[idx_vm.at[pl.ds(j * 128, 128)]]
            pltpu.async_copy(src, rows_vm, sem).wait()
            pltpu.sync_copy(rows_vm, odst.at[pl.ds(base + j * 128, 128)])
            return 0

        lax.fori_loop(0, T // 128, g_step, 0)

    return k(rank_k, outs, lgs)


def _k6_body(o00, o01, o10, o11, l00, l01, l10, l11, f_ref):
    halves = []
    for (oa, ob, la, lb) in ((o00, o01, l00, l01), (o10, o11, l10, l11)):
        l0 = la[...]
        l1 = lb[...]
        m = jnp.maximum(l0, l1)
        e0 = jnp.exp(l0 - m)
        e1 = jnp.exp(l1 - m)
        halves.append((oa[...] * e0 + ob[...] * e1) / (e0 + e1))
    f_ref[0] = jnp.concatenate(halves, axis=1)


def _combine(out_u, lg_u):
    ospec = lambda r: pl.BlockSpec((T, HS), lambda g, _r=r: (4 * g + _r, 0))
    lspec = lambda r: pl.BlockSpec((T, 1), lambda g, _r=r: (4 * g + _r, 0))
    return pl.pallas_call(
        _k6_body,
        grid=(NH // 2,),
        in_specs=[ospec(0), ospec(1), ospec(2), ospec(3),
                  lspec(0), lspec(1), lspec(2), lspec(3)],
        out_specs=pl.BlockSpec((1, T, 2 * HS), lambda g: (0, 0, g)),
        out_shape=jax.ShapeDtypeStruct((1, T, HIDDEN), jnp.float32),
    )(out_u, out_u, out_u, out_u, lg_u, lg_u, lg_u, lg_u)


def kernel(decoder_states, hidden_states, W_qk, W_v):
    ds = decoder_states[0]
    hs = hidden_states[0]
    wq = W_qk.reshape(HIDDEN, NH, HS).transpose(1, 0, 2)
    wv = W_v.reshape(HIDDEN, NH, HS).transpose(1, 0, 2)
    qtab = _project(ds, wq)
    ktab = _project(hs, wq)
    vtab = _project(hs, wv)

    rot = jax.random.normal(jax.random.key(HASH_SEED),
                            (NH, HS, NHASH, NB // 2), jnp.float32)
    rot4 = rot.transpose(0, 2, 1, 3)
    rank_q = _buckets_ranks(qtab, rot4)
    rank_k = _buckets_ranks(ktab, rot4)

    posq, poskl, qs, ks, vs = _sc_sort_gather(rank_q, rank_k,
                                              qtab, ktab, vtab)
    out_s, lg_s = _attention(qs, ks, vs, posq, poskl)
    out_u, lg_u = _sc_unsort(rank_k, out_s, lg_s)
    return _combine(out_u, lg_u)


# speedup vs baseline: 1.3485x; 1.3485x over previous
import functools

import jax
import jax.numpy as jnp
from jax import lax
from jax.experimental import pallas as pl
from jax.experimental.pallas import tpu as pltpu
from jax.experimental.pallas import tpu_sc as plsc

T = 4096
HIDDEN = 1024
NH = 16
HS = 64
CHUNK = 64
NHASH = 2
NB = 128
HASH_SEED = 1234
MASKVAL = -1e5
S = NHASH * T
NCH = S // CHUNK
R = NH * S


def _k1_body(x_ref, w_ref, o_ref):
    o_ref[...] = jnp.dot(x_ref[...], w_ref[0],
                         preferred_element_type=jnp.float32)


def _project(x, w_heads):
    return pl.pallas_call(
        _k1_body,
        grid=(4, NH),
        in_specs=[
            pl.BlockSpec((1024, HIDDEN), lambda i, h: (i, 0)),
            pl.BlockSpec((1, HIDDEN, 2 * HS), lambda i, h: (h, 0, 0)),
        ],
        out_specs=pl.BlockSpec((1024, 2 * HS), lambda i, h: (h * 4 + i, 0)),
        out_shape=jax.ShapeDtypeStruct((NH * T, 2 * HS), jnp.float32),
    )(x, w_heads)


def _k2_body(x_ref, rot_ref, rank_ref):
    x = x_ref[...]
    rot = rot_ref[0, 0]
    r = jnp.dot(x, rot, preferred_element_type=jnp.float32)

    ii = lax.broadcasted_iota(jnp.int32, (T, HS), 1).astype(jnp.float32)
    mx = jnp.max(r, axis=1, keepdims=True)
    mn = jnp.min(r, axis=1, keepdims=True)
    a1 = jnp.min(jnp.where(r == mx, ii, 64.0), axis=1, keepdims=True)
    a2 = jnp.min(jnp.where(r == mn, ii, 64.0), axis=1, keepdims=True) + 64.0
    bucket = jnp.where(mx >= -mn, a1, a2)

    bi = lax.broadcasted_iota(jnp.int32, (128, NB), 1).astype(jnp.float32)
    rows = lax.broadcasted_iota(jnp.int32, (128, 128), 0)
    cols = lax.broadcasted_iota(jnp.int32, (128, 128), 1)
    lstrict = (rows > cols).astype(jnp.float32)
    ustrict = (rows < cols).astype(jnp.float32)

    run = jnp.zeros((1, NB), jnp.float32)
    bases = []
    for blk in range(32):
        ob = (bucket[blk * 128:(blk + 1) * 128] == bi).astype(jnp.float32)
        bases.append(run)
        run = run + jnp.sum(ob, axis=0, keepdims=True)
    binbase = jnp.dot(run, ustrict, preferred_element_type=jnp.float32)

    for blk in range(32):
        ob = (bucket[blk * 128:(blk + 1) * 128] == bi).astype(jnp.float32)
        pb = jnp.dot(lstrict, ob, preferred_element_type=jnp.float32)
        base = binbase + bases[blk]
        rk = jnp.sum(ob * base, axis=1, keepdims=True) + \
             jnp.sum(ob * pb, axis=1, keepdims=True)
        rank_ref[0, 0, blk * 128:(blk + 1) * 128, :] = rk.astype(jnp.int32)


def _buckets_ranks(tab, rot4):
    return pl.pallas_call(
        _k2_body,
        grid=(NH, NHASH),
        in_specs=[
            pl.BlockSpec((T, 2 * HS), lambda h, a: (h, 0)),
            pl.BlockSpec((1, 1, 2 * HS, HS), lambda h, a: (h, a, 0, 0)),
        ],
        out_specs=pl.BlockSpec((1, 1, T, 1), lambda h, a: (h, a, 0, 0)),
        out_shape=jax.ShapeDtypeStruct((NH, NHASH, T, 1), jnp.int32),
    )(tab, rot4)


_SC_MESH = plsc.VectorSubcoreMesh(core_axis_name="c", subcore_axis_name="s")
_SC_PARAMS = pltpu.CompilerParams(needs_layout_passes=False,
                                  use_tc_tiling_on_sc=False)
_Z16 = lambda: jnp.zeros((16,), jnp.int32)


def _sc_sort_gather(rank_q, rank_k, qtab, kvtab):
    @functools.partial(
        pl.kernel,
        mesh=_SC_MESH,
        compiler_params=_SC_PARAMS,
        out_type=[
            jax.ShapeDtypeStruct((R, 1), jnp.float32),
            jax.ShapeDtypeStruct((NH, NCH, 1, CHUNK), jnp.float32),
            jax.ShapeDtypeStruct((R, 2 * HS), jnp.float32),
            jax.ShapeDtypeStruct((R, 2 * HS), jnp.float32),
        ],
        scratch_types=[
            pltpu.VMEM((T, 1), jnp.int32),
            pltpu.VMEM((T,), jnp.int32),
            pltpu.VMEM((T,), jnp.int32),
            pltpu.VMEM((T, 1), jnp.float32),
            pltpu.VMEM((CHUNK, 1, CHUNK), jnp.float32),
            pltpu.VMEM((128, 2 * HS), jnp.float32),
            pltpu.SemaphoreType.DMA,
        ],
    )
    def k(rq, rk, qt, kvt, posq, poskl, qs, kvs,
          rank_vm, idxq_vm, idxk_vm, pcol_vm, plane_vm, rows_vm, sem):
        wid = lax.axis_index("s") * 2 + lax.axis_index("c")
        h = wid // 2
        a = wid % 2

        def inv_q(j, _):
            seq = j * 16 + lax.iota(jnp.int32, 16)
            rk16 = plsc.load_gather(rank_vm, [seq, _Z16()])
            plsc.store_scatter(idxq_vm, [rk16], seq + h * T)
            plsc.store_scatter(pcol_vm, [rk16, _Z16()],
                               seq.astype(jnp.float32))
            return 0

        def inv_k(j, _):
            seq = j * 16 + lax.iota(jnp.int32, 16)
            rk16 = plsc.load_gather(rank_vm, [seq, _Z16()])
            plsc.store_scatter(idxk_vm, [rk16], seq + h * T)
            plsc.store_scatter(
                plane_vm,
                [lax.shift_right_logical(rk16, 6), _Z16(),
                 lax.bitwise_and(rk16, 63)],
                seq.astype(jnp.float32))
            return 0

        def gather(tab, idx_vm, dst):
            def g_step(j, _):
                src = tab.at[idx_vm.at[pl.ds(j * 128, 128)]]
                pltpu.async_copy(src, rows_vm, sem).wait()
                pltpu.sync_copy(rows_vm,
                                dst.at[pl.ds(wid * T + j * 128, 128)])
                return 0

            lax.fori_loop(0, T // 128, g_step, 0)

        pltpu.sync_copy(rq.at[h, a], rank_vm)
        lax.fori_loop(0, T // 16, inv_q, 0)
        pltpu.sync_copy(pcol_vm, posq.at[pl.ds(wid * T, T)])
        pltpu.sync_copy(rk.at[h, a], rank_vm)
        lax.fori_loop(0, T // 16, inv_k, 0)
        pltpu.sync_copy(plane_vm, poskl.at[h, pl.ds(a * CHUNK, CHUNK)])
        gather(qt, idxq_vm, qs)
        gather(kvt, idxk_vm, kvs)

    return k(rank_q, rank_k, qtab, kvtab)


def _k4_body(qs_ref, kvs_ref, pq_ref, pkl_ref,
             kvw_ref, pkw_ref,
             out_ref, pk_scr, pp_scr):
    g = pl.program_id(1)
    lane = lax.broadcasted_iota(jnp.int32, (64, 2 * HS), 1)
    kmask = lane < HS

    def norm_kv(rows):
        kk = jnp.where(kmask, rows, 0.0)
        var = jnp.sum(kk * kk, axis=1, keepdims=True) * (1.0 / HS)
        scale = lax.rsqrt(var + 1e-6) * (HS ** -0.5)
        return rows * jnp.where(kmask, scale, 1.0)

    @pl.when(g == 0)
    def _init():
        pk_scr[...] = norm_kv(kvw_ref[...])
        pp_scr[...] = pkw_ref[0, 0]

    kvprev = pk_scr[...]
    pprev = pp_scr[...]
    nt = (((1,), (1,)), ((), ()))
    for j in range(16):
        qc = qs_ref[j * 64:(j + 1) * 64, :]
        kvn = norm_kv(kvs_ref[j * 64:(j + 1) * 64, :])
        pkc = pkl_ref[0, j, :, :]
        pqc = pq_ref[j * 64:(j + 1) * 64, :]
        d0 = lax.dot_general(qc, kvprev, nt,
                             preferred_element_type=jnp.float32)
        d1 = lax.dot_general(qc, kvn, nt,
                             preferred_element_type=jnp.float32)
        d0 = jnp.where(pqc != pprev, d0, MASKVAL)
        d1 = jnp.where(pqc != pkc, d1, MASKVAL)
        m = jnp.maximum(jnp.max(d0, axis=1, keepdims=True),
                        jnp.max(d1, axis=1, keepdims=True))
        e0 = jnp.exp(d0 - m)
        e1 = jnp.exp(d1 - m)
        s = jnp.sum(e0, axis=1, keepdims=True) + \
            jnp.sum(e1, axis=1, keepdims=True)
        o = (jnp.dot(e0, kvprev, preferred_element_type=jnp.float32) +
             jnp.dot(e1, kvn, preferred_element_type=jnp.float32)) / s
        l = m + jnp.log(s)
        out_ref[j * 64:(j + 1) * 64, :] = jnp.where(kmask, l, o)
        kvprev, pprev = kvn, pkc
    pk_scr[...] = kvprev
    pp_scr[...] = pprev


def _attention(qs, kvs, pq, pkl):
    G = S // 8
    return pl.pallas_call(
        _k4_body,
        grid=(NH, 8),
        in_specs=[
            pl.BlockSpec((G, 2 * HS), lambda h, g: (h * 8 + g, 0)),
            pl.BlockSpec((G, 2 * HS), lambda h, g: (h * 8 + g, 0)),
            pl.BlockSpec((G, 1), lambda h, g: (h * 8 + g, 0)),
            pl.BlockSpec((1, 16, 1, HS), lambda h, g: (h, g, 0, 0)),
            pl.BlockSpec((64, 2 * HS), lambda h, g: (h * NCH + NCH - 1, 0)),
            pl.BlockSpec((1, 1, 1, HS), lambda h, g: (h, NCH - 1, 0, 0)),
        ],
        out_specs=pl.BlockSpec((G, 2 * HS), lambda h, g: (h * 8 + g, 0)),
        out_shape=jax.ShapeDtypeStruct((R, 2 * HS), jnp.float32),
        scratch_shapes=[
            pltpu.VMEM((64, 2 * HS), jnp.float32),
            pltpu.VMEM((1, HS), jnp.float32),
        ],
    )(qs, kvs, pq, pkl, kvs, pkl)


def _sc_unsort(rank_k, outs):
    @functools.partial(
        pl.kernel,
        mesh=_SC_MESH,
        compiler_params=_SC_PARAMS,
        out_type=jax.ShapeDtypeStruct((R, 2 * HS), jnp.float32),
        scratch_types=[
            pltpu.VMEM((T, 1), jnp.int32),
            pltpu.VMEM((T,), jnp.int32),
            pltpu.VMEM((128, 2 * HS), jnp.float32),
            pltpu.SemaphoreType.DMA,
        ],
    )
    def k(rk, osrc, odst, rank_vm, idx_vm, rows_vm, sem):
        wid = lax.axis_index("s") * 2 + lax.axis_index("c")
        h = wid // 2
        a = wid % 2
        base = wid * T
        pltpu.sync_copy(rk.at[h, a], rank_vm)

        def mk_idx(j, _):
            seq = j * 16 + lax.iota(jnp.int32, 16)
            r16 = plsc.load_gather(rank_vm, [seq, _Z16()])
            idx_vm[pl.ds(j * 16, 16)] = r16 + base
            return 0

        lax.fori_loop(0, T // 16, mk_idx, 0)

        def g_step(j, _):
            src = osrc.at[idx_vm.at[pl.ds(j * 128, 128)]]
            pltpu.async_copy(src, rows_vm, sem).wait()
            pltpu.sync_copy(rows_vm, odst.at[pl.ds(base + j * 128, 128)])
            return 0

        lax.fori_loop(0, T // 128, g_step, 0)

    return k(rank_k, outs)


def _k6_body(x00, x01, x10, x11, f_ref):
    halves = []
    for (xa, xb) in ((x00, x01), (x10, x11)):
        va = xa[...]
        vb = xb[...]
        l0 = va[:, 0:1]
        l1 = vb[:, 0:1]
        m = jnp.maximum(l0, l1)
        e0 = jnp.exp(l0 - m)
        e1 = jnp.exp(l1 - m)
        o = (va * e0 + vb * e1) / (e0 + e1)
        halves.append(o[:, HS:])
    f_ref[0] = jnp.concatenate(halves, axis=1)


def _combine(out_u):
    spec = lambda r: pl.BlockSpec((T, 2 * HS), lambda g, _r=r: (4 * g + _r, 0))
    return pl.pallas_call(
        _k6_body,
        grid=(NH // 2,),
        in_specs=[spec(0), spec(1), spec(2), spec(3)],
        out_specs=pl.BlockSpec((1, T, 2 * HS), lambda g: (0, 0, g)),
        out_shape=jax.ShapeDtypeStruct((1, T, HIDDEN), jnp.float32),
    )(out_u, out_u, out_u, out_u)


def kernel(decoder_states, hidden_states, W_qk, W_v):
    ds = decoder_states[0]
    hs = hidden_states[0]
    wq = W_qk.reshape(HIDDEN, NH, HS).transpose(1, 0, 2)
    wv = W_v.reshape(HIDDEN, NH, HS).transpose(1, 0, 2)
    zeros = jnp.zeros_like(wq)
    wq128 = jnp.concatenate([wq, zeros], axis=-1)
    wkv = jnp.concatenate([wq, wv], axis=-1)
    qtab = _project(ds, wq128)
    kvtab = _project(hs, wkv)

    rot = jax.random.normal(jax.random.key(HASH_SEED),
                            (NH, HS, NHASH, NB // 2), jnp.float32)
    rot4 = rot.transpose(0, 2, 1, 3)
    rot4 = jnp.concatenate([rot4, jnp.zeros_like(rot4)], axis=2)
    rank_q = _buckets_ranks(qtab, rot4)
    rank_k = _buckets_ranks(kvtab, rot4)

    posq, poskl, qs, kvs = _sc_sort_gather(rank_q, rank_k, qtab, kvtab)
    out_s = _attention(qs, kvs, posq, poskl)
    out_u = _sc_unsort(rank_k, out_s)
    return _combine(out_u)


# speedup vs baseline: 1.6293x; 1.2083x over previous
import functools

import jax
import jax.numpy as jnp
from jax import lax
from jax.experimental import pallas as pl
from jax.experimental.pallas import tpu as pltpu
from jax.experimental.pallas import tpu_sc as plsc

T = 4096
HIDDEN = 1024
NH = 16
HS = 64
CHUNK = 64
NHASH = 2
NB = 128
HASH_SEED = 1234
MASKVAL = -1e5
S = NHASH * T
NCH = S // CHUNK
R = NH * S


def _k1q_body(x_ref, w_ref, o_ref):
    y = jnp.dot(x_ref[...], w_ref[0], preferred_element_type=jnp.float32)
    i = pl.program_id(0)
    pos = (lax.broadcasted_iota(jnp.int32, (1024, 1), 0)
           + i * 1024).astype(jnp.float32)
    o_ref[...] = jnp.concatenate(
        [y, pos, jnp.zeros((1024, HS - 1), jnp.float32)], axis=1)


def _k1kv_body(x_ref, w_ref, o_ref):
    o_ref[...] = jnp.dot(x_ref[...], w_ref[0],
                         preferred_element_type=jnp.float32)


def _project(x, w_heads, body):
    W = w_heads.shape[-1]
    return pl.pallas_call(
        body,
        grid=(4, NH),
        in_specs=[
            pl.BlockSpec((1024, HIDDEN), lambda i, h: (i, 0)),
            pl.BlockSpec((1, HIDDEN, W), lambda i, h: (h, 0, 0)),
        ],
        out_specs=pl.BlockSpec((1024, 2 * HS), lambda i, h: (h * 4 + i, 0)),
        out_shape=jax.ShapeDtypeStruct((NH * T, 2 * HS), jnp.float32),
    )(x, w_heads)


def _k2_body(x_ref, rot_ref, rank_ref):
    x = x_ref[...]
    rot = rot_ref[0, 0]
    r = jnp.dot(x, rot, preferred_element_type=jnp.float32)

    ii = lax.broadcasted_iota(jnp.int32, (T, HS), 1).astype(jnp.float32)
    mx = jnp.max(r, axis=1, keepdims=True)
    mn = jnp.min(r, axis=1, keepdims=True)
    a1 = jnp.min(jnp.where(r == mx, ii, 64.0), axis=1, keepdims=True)
    a2 = jnp.min(jnp.where(r == mn, ii, 64.0), axis=1, keepdims=True) + 64.0
    bucket = jnp.where(mx >= -mn, a1, a2)

    bi = lax.broadcasted_iota(jnp.int32, (128, NB), 1).astype(jnp.float32)
    rows = lax.broadcasted_iota(jnp.int32, (128, 128), 0)
    cols = lax.broadcasted_iota(jnp.int32, (128, 128), 1)
    lstrict = (rows > cols).astype(jnp.float32)
    ustrict = (rows < cols).astype(jnp.float32)

    run = jnp.zeros((1, NB), jnp.float32)
    bases = []
    for blk in range(32):
        ob = (bucket[blk * 128:(blk + 1) * 128] == bi).astype(jnp.float32)
        bases.append(run)
        run = run + jnp.sum(ob, axis=0, keepdims=True)
    binbase = jnp.dot(run, ustrict, preferred_element_type=jnp.float32)

    for blk in range(32):
        ob = (bucket[blk * 128:(blk + 1) * 128] == bi).astype(jnp.float32)
        pb = jnp.dot(lstrict, ob, preferred_element_type=jnp.float32)
        base = binbase + bases[blk]
        rk = jnp.sum(ob * base, axis=1, keepdims=True) + \
             jnp.sum(ob * pb, axis=1, keepdims=True)
        rank_ref[0, 0, blk * 128:(blk + 1) * 128, :] = rk.astype(jnp.int32)


def _buckets_ranks(tab, rot4):
    return pl.pallas_call(
        _k2_body,
        grid=(NH, NHASH),
        in_specs=[
            pl.BlockSpec((T, 2 * HS), lambda h, a: (h, 0)),
            pl.BlockSpec((1, 1, 2 * HS, HS), lambda h, a: (h, a, 0, 0)),
        ],
        out_specs=pl.BlockSpec((1, 1, T, 1), lambda h, a: (h, a, 0, 0)),
        out_shape=jax.ShapeDtypeStruct((NH, NHASH, T, 1), jnp.int32),
    )(tab, rot4)


def _sc_mesh():
    return plsc.VectorSubcoreMesh(core_axis_name="c", subcore_axis_name="s")
_SC_PARAMS = pltpu.CompilerParams(needs_layout_passes=False,
                                  use_tc_tiling_on_sc=False)
_Z16 = lambda: jnp.zeros((16,), jnp.int32)


def _sc_sort_gather(rank_q, rank_k, qtab, kvtab):
    @functools.partial(
        pl.kernel,
        mesh=_sc_mesh(),
        compiler_params=_SC_PARAMS,
        out_type=[
            jax.ShapeDtypeStruct((NH, NCH, 1, CHUNK), jnp.float32),
            jax.ShapeDtypeStruct((R, 2 * HS), jnp.float32),
            jax.ShapeDtypeStruct((R, 2 * HS), jnp.float32),
        ],
        scratch_types=[
            pltpu.VMEM((T, 1), jnp.int32),
            pltpu.VMEM((T,), jnp.int32),
            pltpu.VMEM((T,), jnp.int32),
            pltpu.VMEM((CHUNK, 1, CHUNK), jnp.float32),
            pltpu.VMEM((128, 2 * HS), jnp.float32),
            pltpu.VMEM((128, 2 * HS), jnp.float32),
            pltpu.SemaphoreType.DMA,
            pltpu.SemaphoreType.DMA,
        ],
    )
    def k(rq, rk, qt, kvt, poskl, qs, kvs,
          rank_vm, idxq_vm, idxk_vm, plane_vm, rows_a, rows_b, sem_a, sem_b):
        wid = lax.axis_index("s") * 2 + lax.axis_index("c")
        h = wid // 2
        a = wid % 2

        def inv_q(j, _):
            seq = j * 16 + lax.iota(jnp.int32, 16)
            rk16 = plsc.load_gather(rank_vm, [seq, _Z16()])
            plsc.store_scatter(idxq_vm, [rk16], seq + h * T)
            return 0

        def inv_k(j, _):
            seq = j * 16 + lax.iota(jnp.int32, 16)
            rk16 = plsc.load_gather(rank_vm, [seq, _Z16()])
            plsc.store_scatter(idxk_vm, [rk16], seq + h * T)
            plsc.store_scatter(
                plane_vm,
                [lax.shift_right_logical(rk16, 6), _Z16(),
                 lax.bitwise_and(rk16, 63)],
                seq.astype(jnp.float32))
            return 0

        def gather(tab, idx_vm, dst):
            def g_step(i, _):
                j = i * 2
                cpa = pltpu.async_copy(
                    tab.at[idx_vm.at[pl.ds(j * 128, 128)]], rows_a, sem_a)
                cpb = pltpu.async_copy(
                    tab.at[idx_vm.at[pl.ds(j * 128 + 128, 128)]],
                    rows_b, sem_b)
                cpa.wait()
                pltpu.sync_copy(rows_a,
                                dst.at[pl.ds(wid * T + j * 128, 128)])
                cpb.wait()
                pltpu.sync_copy(rows_b,
                                dst.at[pl.ds(wid * T + j * 128 + 128, 128)])
                return 0

            lax.fori_loop(0, T // 256, g_step, 0)

        pltpu.sync_copy(rq.at[h, a], rank_vm)
        lax.fori_loop(0, T // 16, inv_q, 0)
        pltpu.sync_copy(rk.at[h, a], rank_vm)
        lax.fori_loop(0, T // 16, inv_k, 0)
        pltpu.sync_copy(plane_vm, poskl.at[h, pl.ds(a * CHUNK, CHUNK)])
        gather(qt, idxq_vm, qs)
        gather(kvt, idxk_vm, kvs)

    return k(rank_q, rank_k, qtab, kvtab)


def _k4_body(qs_ref, kvs_ref, pkl_ref, kvw_ref, pkw_ref,
             out_ref, pk_scr, pp_scr):
    g = pl.program_id(1)
    lane = lax.broadcasted_iota(jnp.int32, (64, 2 * HS), 1)
    kmask = lane < HS
    kmaskf = kmask.astype(jnp.float32)
    eselm = (lane == HS).astype(jnp.float32)

    def norm_kv(rows):
        kk = rows * kmaskf
        var = jnp.sum(kk * kk, axis=1, keepdims=True) * (1.0 / HS)
        scale = lax.rsqrt(var + 1e-6) * (HS ** -0.5)
        return rows * jnp.where(kmask, scale, 1.0)

    @pl.when(g == 0)
    def _init():
        pk_scr[...] = norm_kv(kvw_ref[...])
        pp_scr[...] = pkw_ref[0, 0]

    kvprev = pk_scr[...]
    pprev = pp_scr[...]
    nt = (((1,), (1,)), ((), ()))
    for j in range(16):
        qc = qs_ref[j * 64:(j + 1) * 64, :]
        qm = qc * kmaskf
        kvn = norm_kv(kvs_ref[j * 64:(j + 1) * 64, :])
        pkc = pkl_ref[0, j, :, :]
        pqc = jnp.sum(qc * eselm, axis=1, keepdims=True)
        d0 = lax.dot_general(qm, kvprev, nt,
                             preferred_element_type=jnp.float32)
        d1 = lax.dot_general(qm, kvn, nt,
                             preferred_element_type=jnp.float32)
        d0 = jnp.where(pqc != pprev, d0, MASKVAL)
        d1 = jnp.where(pqc != pkc, d1, MASKVAL)
        e0 = jnp.exp(d0)
        e1 = jnp.exp(d1)
        s = jnp.sum(e0, axis=1, keepdims=True) + \
            jnp.sum(e1, axis=1, keepdims=True)
        inv = 1.0 / s
        o = (jnp.dot(e0, kvprev, preferred_element_type=jnp.float32) +
             jnp.dot(e1, kvn, preferred_element_type=jnp.float32)) * inv
        l = jnp.log(s)
        out_ref[j * 64:(j + 1) * 64, :] = jnp.where(kmask, l, o)
        kvprev, pprev = kvn, pkc
    pk_scr[...] = kvprev
    pp_scr[...] = pprev


def _attention(qs, kvs, pkl):
    G = S // 8
    return pl.pallas_call(
        _k4_body,
        grid=(NH, 8),
        in_specs=[
            pl.BlockSpec((G, 2 * HS), lambda h, g: (h * 8 + g, 0)),
            pl.BlockSpec((G, 2 * HS), lambda h, g: (h * 8 + g, 0)),
            pl.BlockSpec((1, 16, 1, HS), lambda h, g: (h, g, 0, 0)),
            pl.BlockSpec((64, 2 * HS), lambda h, g: (h * NCH + NCH - 1, 0)),
            pl.BlockSpec((1, 1, 1, HS), lambda h, g: (h, NCH - 1, 0, 0)),
        ],
        out_specs=pl.BlockSpec((G, 2 * HS), lambda h, g: (h * 8 + g, 0)),
        out_shape=jax.ShapeDtypeStruct((R, 2 * HS), jnp.float32),
        scratch_shapes=[
            pltpu.VMEM((64, 2 * HS), jnp.float32),
            pltpu.VMEM((1, HS), jnp.float32),
        ],
    )(qs, kvs, pkl, kvs, pkl)


def _sc_unsort(rank_k, outs):
    @functools.partial(
        pl.kernel,
        mesh=_sc_mesh(),
        compiler_params=_SC_PARAMS,
        out_type=jax.ShapeDtypeStruct((R, 2 * HS), jnp.float32),
        scratch_types=[
            pltpu.VMEM((T, 1), jnp.int32),
            pltpu.VMEM((T,), jnp.int32),
            pltpu.VMEM((128, 2 * HS), jnp.float32),
            pltpu.VMEM((128, 2 * HS), jnp.float32),
            pltpu.SemaphoreType.DMA,
            pltpu.SemaphoreType.DMA,
        ],
    )
    def k(rk, osrc, odst, rank_vm, idx_vm, rows_a, rows_b, sem_a, sem_b):
        wid = lax.axis_index("s") * 2 + lax.axis_index("c")
        h = wid // 2
        a = wid % 2
        base = wid * T
        pltpu.sync_copy(rk.at[h, a], rank_vm)

        def mk_idx(j, _):
            seq = j * 16 + lax.iota(jnp.int32, 16)
            r16 = plsc.load_gather(rank_vm, [seq, _Z16()])
            idx_vm[pl.ds(j * 16, 16)] = r16 + base
            return 0

        lax.fori_loop(0, T // 16, mk_idx, 0)

        def g_step(i, _):
            j = i * 2
            cpa = pltpu.async_copy(
                osrc.at[idx_vm.at[pl.ds(j * 128, 128)]], rows_a, sem_a)
            cpb = pltpu.async_copy(
                osrc.at[idx_vm.at[pl.ds(j * 128 + 128, 128)]], rows_b, sem_b)
            cpa.wait()
            pltpu.sync_copy(rows_a, odst.at[pl.ds(base + j * 128, 128)])
            cpb.wait()
            pltpu.sync_copy(rows_b, odst.at[pl.ds(base + j * 128 + 128, 128)])
            return 0

        lax.fori_loop(0, T // 256, g_step, 0)

    return k(rank_k, outs)


def _k6_body(x00, x01, x10, x11, f_ref):
    halves = []
    for (xa, xb) in ((x00, x01), (x10, x11)):
        va = xa[...]
        vb = xb[...]
        l0 = va[:, 0:1]
        l1 = vb[:, 0:1]
        m = jnp.maximum(l0, l1)
        e0 = jnp.exp(l0 - m)
        e1 = jnp.exp(l1 - m)
        o = (va * e0 + vb * e1) / (e0 + e1)
        halves.append(o[:, HS:])
    f_ref[0] = jnp.concatenate(halves, axis=1)


def _combine(out_u):
    spec = lambda r: pl.BlockSpec((T, 2 * HS), lambda g, _r=r: (4 * g + _r, 0))
    return pl.pallas_call(
        _k6_body,
        grid=(NH // 2,),
        in_specs=[spec(0), spec(1), spec(2), spec(3)],
        out_specs=pl.BlockSpec((1, T, 2 * HS), lambda g: (0, 0, g)),
        out_shape=jax.ShapeDtypeStruct((1, T, HIDDEN), jnp.float32),
    )(out_u, out_u, out_u, out_u)


def kernel(decoder_states, hidden_states, W_qk, W_v):
    ds = decoder_states[0]
    hs = hidden_states[0]
    wq = W_qk.reshape(HIDDEN, NH, HS).transpose(1, 0, 2)
    wv = W_v.reshape(HIDDEN, NH, HS).transpose(1, 0, 2)
    wkv = jnp.concatenate([wq, wv], axis=-1)
    qtab = _project(ds, wq, _k1q_body)
    kvtab = _project(hs, wkv, _k1kv_body)

    rot = jax.random.normal(jax.random.key(HASH_SEED),
                            (NH, HS, NHASH, NB // 2), jnp.float32)
    rot4 = rot.transpose(0, 2, 1, 3)
    rot4 = jnp.concatenate([rot4, jnp.zeros_like(rot4)], axis=2)
    rank_q = _buckets_ranks(qtab, rot4)
    rank_k = _buckets_ranks(kvtab, rot4)

    poskl, qs, kvs = _sc_sort_gather(rank_q, rank_k, qtab, kvtab)
    out_s = _attention(qs, kvs, poskl)
    out_u = _sc_unsort(rank_k, out_s)
    return _combine(out_u)


# speedup vs baseline: 1.6522x; 1.0141x over previous
import functools

import jax
import jax.numpy as jnp
from jax import lax
from jax.experimental import pallas as pl
from jax.experimental.pallas import tpu as pltpu
from jax.experimental.pallas import tpu_sc as plsc

T = 4096
HIDDEN = 1024
NH = 16
HS = 64
CHUNK = 64
NHASH = 2
NB = 128
HASH_SEED = 1234
MASKVAL = -1e5
S = NHASH * T
NCH = S // CHUNK
R = NH * S


def _k1q_body(x_ref, w_ref, o_ref):
    y = jnp.dot(x_ref[...], w_ref[0], preferred_element_type=jnp.float32)
    i = pl.program_id(0)
    pos = (lax.broadcasted_iota(jnp.int32, (1024, 1), 0)
           + i * 1024).astype(jnp.float32)
    o_ref[...] = jnp.concatenate(
        [y, pos, jnp.zeros((1024, HS - 1), jnp.float32)], axis=1)


def _k1kv_body(x_ref, w_ref, o_ref):
    o_ref[...] = jnp.dot(x_ref[...], w_ref[0],
                         preferred_element_type=jnp.float32)


def _project(x, w_heads, body):
    W = w_heads.shape[-1]
    return pl.pallas_call(
        body,
        grid=(4, NH),
        in_specs=[
            pl.BlockSpec((1024, HIDDEN), lambda i, h: (i, 0)),
            pl.BlockSpec((1, HIDDEN, W), lambda i, h: (h, 0, 0)),
        ],
        out_specs=pl.BlockSpec((1024, 2 * HS), lambda i, h: (h * 4 + i, 0)),
        out_shape=jax.ShapeDtypeStruct((NH * T, 2 * HS), jnp.float32),
    )(x, w_heads)


def _k2_body(x_ref, rot_ref, rank_ref, ob_scr):
    x = x_ref[...]
    rot = rot_ref[0, 0]
    r = jnp.dot(x, rot, preferred_element_type=jnp.float32)

    ii = lax.broadcasted_iota(jnp.int32, (T, HS), 1).astype(jnp.float32)
    mx = jnp.max(r, axis=1, keepdims=True)
    mn = jnp.min(r, axis=1, keepdims=True)
    a1 = jnp.min(jnp.where(r == mx, ii, 64.0), axis=1, keepdims=True)
    a2 = jnp.min(jnp.where(r == mn, ii, 64.0), axis=1, keepdims=True) + 64.0
    bucket = jnp.where(mx >= -mn, a1, a2)

    bi = lax.broadcasted_iota(jnp.int32, (128, NB), 1).astype(jnp.float32)
    rows = lax.broadcasted_iota(jnp.int32, (128, 128), 0)
    cols = lax.broadcasted_iota(jnp.int32, (128, 128), 1)
    lstrict = (rows > cols).astype(jnp.float32)
    ustrict = (rows < cols).astype(jnp.float32)

    ones_row = jnp.ones((1, 128), jnp.float32)
    run = jnp.zeros((1, NB), jnp.float32)
    bases = []
    for blk in range(32):
        ob = (bucket[blk * 128:(blk + 1) * 128] == bi).astype(jnp.float32)
        ob_scr[blk * 128:(blk + 1) * 128, :] = ob
        bases.append(run)
        run = run + jnp.dot(ones_row, ob, preferred_element_type=jnp.float32)
    binbase = jnp.dot(run, ustrict, preferred_element_type=jnp.float32)

    for blk in range(32):
        ob = ob_scr[blk * 128:(blk + 1) * 128, :]
        pb = jnp.dot(lstrict, ob, preferred_element_type=jnp.float32)
        rk = jnp.sum(ob * (pb + binbase + bases[blk]), axis=1, keepdims=True)
        rank_ref[0, 0, blk:blk + 1, :] = \
            lax.transpose(rk, (1, 0)).astype(jnp.int32)


def _buckets_ranks(tab, rot4):
    return pl.pallas_call(
        _k2_body,
        grid=(NH, NHASH),
        in_specs=[
            pl.BlockSpec((T, 2 * HS), lambda h, a: (h, 0)),
            pl.BlockSpec((1, 1, 2 * HS, HS), lambda h, a: (h, a, 0, 0)),
        ],
        out_specs=pl.BlockSpec((1, 1, 32, 128), lambda h, a: (h, a, 0, 0)),
        out_shape=jax.ShapeDtypeStruct((NH, NHASH, 32, 128), jnp.int32),
        scratch_shapes=[pltpu.VMEM((T, 128), jnp.float32)],
    )(tab, rot4)


def _sc_mesh():
    return plsc.VectorSubcoreMesh(core_axis_name="c", subcore_axis_name="s")
_SC_PARAMS = pltpu.CompilerParams(needs_layout_passes=False,
                                  use_tc_tiling_on_sc=False)
_Z16 = lambda: jnp.zeros((16,), jnp.int32)


def _sc_sort_gather(rank_q, rank_k, qtab, kvtab):
    @functools.partial(
        pl.kernel,
        mesh=_sc_mesh(),
        compiler_params=_SC_PARAMS,
        out_type=[
            jax.ShapeDtypeStruct((NH, NCH // 2, 128), jnp.float32),
            jax.ShapeDtypeStruct((R, 2 * HS), jnp.float32),
            jax.ShapeDtypeStruct((R, 2 * HS), jnp.float32),
        ],
        scratch_types=[
            pltpu.VMEM((32, 128), jnp.int32),
            pltpu.VMEM((T,), jnp.int32),
            pltpu.VMEM((T,), jnp.int32),
            pltpu.VMEM((32, 128), jnp.float32),
            pltpu.VMEM((128, 2 * HS), jnp.float32),
            pltpu.VMEM((128, 2 * HS), jnp.float32),
            pltpu.SemaphoreType.DMA,
            pltpu.SemaphoreType.DMA,
        ],
    )
    def k(rq, rk, qt, kvt, poskl, qs, kvs,
          rank_vm, idxq_vm, idxk_vm, plane_vm, rows_a, rows_b, sem_a, sem_b):
        wid = lax.axis_index("s") * 2 + lax.axis_index("c")
        h = wid // 2
        a = wid % 2

        def inv_q(i, _):
            for l in range(8):
                seq = i * 128 + l * 16 + lax.iota(jnp.int32, 16)
                rk16 = rank_vm[i, l * 16:(l + 1) * 16]
                plsc.store_scatter(idxq_vm, [rk16], seq + h * T)
            return 0

        def inv_k(i, _):
            for l in range(8):
                seq = i * 128 + l * 16 + lax.iota(jnp.int32, 16)
                rk16 = rank_vm[i, l * 16:(l + 1) * 16]
                plsc.store_scatter(idxk_vm, [rk16], seq + h * T)
                row = lax.shift_right_logical(rk16, 7)
                lanei = (lax.bitwise_and(lax.shift_right_logical(rk16, 6), 1)
                         * 64 + lax.bitwise_and(rk16, 63))
                plsc.store_scatter(plane_vm, [row, lanei],
                                   seq.astype(jnp.float32))
            return 0

        def gather(tab, idx_vm, dst):
            def g_step(i, _):
                j = i * 2
                cpa = pltpu.async_copy(
                    tab.at[idx_vm.at[pl.ds(j * 128, 128)]], rows_a, sem_a)
                cpb = pltpu.async_copy(
                    tab.at[idx_vm.at[pl.ds(j * 128 + 128, 128)]],
                    rows_b, sem_b)
                cpa.wait()
                pltpu.sync_copy(rows_a,
                                dst.at[pl.ds(wid * T + j * 128, 128)])
                cpb.wait()
                pltpu.sync_copy(rows_b,
                                dst.at[pl.ds(wid * T + j * 128 + 128, 128)])
                return 0

            lax.fori_loop(0, T // 256, g_step, 0)

        pltpu.sync_copy(rq.at[h, a], rank_vm)
        lax.fori_loop(0, 32, inv_q, 0)
        pltpu.sync_copy(rk.at[h, a], rank_vm)
        lax.fori_loop(0, 32, inv_k, 0)
        pltpu.sync_copy(plane_vm, poskl.at[h, pl.ds(a * 32, 32)])
        gather(qt, idxq_vm, qs)
        gather(kvt, idxk_vm, kvs)

    return k(rank_q, rank_k, qtab, kvtab)


def _k4_body(qs_ref, kvs_ref, pkl_ref, kvw_ref, pkw_ref,
             out_ref, pk_scr, pp_scr):
    g = pl.program_id(1)
    lane = lax.broadcasted_iota(jnp.int32, (64, 2 * HS), 1)
    kmask = lane < HS
    kmaskf = kmask.astype(jnp.float32)
    eselm = (lane == HS).astype(jnp.float32)

    def norm_kv(rows):
        kk = rows * kmaskf
        var = jnp.sum(kk * kk, axis=1, keepdims=True) * (1.0 / HS)
        scale = lax.rsqrt(var + 1e-6) * (HS ** -0.5)
        return rows * jnp.where(kmask, scale, 1.0)

    @pl.when(g == 0)
    def _init():
        pk_scr[...] = norm_kv(kvw_ref[...])
        pp_scr[...] = pkw_ref[0, 7:8, :][:, HS:]

    kvprev = pk_scr[...]
    pprev = pp_scr[...]
    nt = (((1,), (1,)), ((), ()))
    for j in range(16):
        qc = qs_ref[j * 64:(j + 1) * 64, :]
        qm = qc * kmaskf
        kvn = norm_kv(kvs_ref[j * 64:(j + 1) * 64, :])
        pkrow = pkl_ref[0, j // 2:j // 2 + 1, :]
        pkc = pkrow[:, HS:] if j % 2 else pkrow[:, :HS]
        pqc = jnp.sum(qc * eselm, axis=1, keepdims=True)
        d0 = lax.dot_general(qm, kvprev, nt,
                             preferred_element_type=jnp.float32)
        d1 = lax.dot_general(qm, kvn, nt,
                             preferred_element_type=jnp.float32)
        d0 = jnp.where(pqc != pprev, d0, MASKVAL)
        d1 = jnp.where(pqc != pkc, d1, MASKVAL)
        e0 = jnp.exp(d0)
        e1 = jnp.exp(d1)
        s = jnp.sum(e0, axis=1, keepdims=True) + \
            jnp.sum(e1, axis=1, keepdims=True)
        inv = 1.0 / s
        o = (jnp.dot(e0, kvprev, preferred_element_type=jnp.float32) +
             jnp.dot(e1, kvn, preferred_element_type=jnp.float32)) * inv
        l = jnp.log(s)
        out_ref[j * 64:(j + 1) * 64, :] = jnp.where(kmask, l, o)
        kvprev, pprev = kvn, pkc
    pk_scr[...] = kvprev
    pp_scr[...] = pprev


def _attention(qs, kvs, pkl):
    G = S // 8
    return pl.pallas_call(
        _k4_body,
        grid=(NH, 8),
        in_specs=[
            pl.BlockSpec((G, 2 * HS), lambda h, g: (h * 8 + g, 0)),
            pl.BlockSpec((G, 2 * HS), lambda h, g: (h * 8 + g, 0)),
            pl.BlockSpec((1, 8, 128), lambda h, g: (h, g, 0)),
            pl.BlockSpec((64, 2 * HS), lambda h, g: (h * NCH + NCH - 1, 0)),
            pl.BlockSpec((1, 8, 128), lambda h, g: (h, 7, 0)),
        ],
        out_specs=pl.BlockSpec((G, 2 * HS), lambda h, g: (h * 8 + g, 0)),
        out_shape=jax.ShapeDtypeStruct((R, 2 * HS), jnp.float32),
        scratch_shapes=[
            pltpu.VMEM((64, 2 * HS), jnp.float32),
            pltpu.VMEM((1, HS), jnp.float32),
        ],
    )(qs, kvs, pkl, kvs, pkl)


def _sc_unsort(rank_k, outs):
    @functools.partial(
        pl.kernel,
        mesh=_sc_mesh(),
        compiler_params=_SC_PARAMS,
        out_type=jax.ShapeDtypeStruct((R, 2 * HS), jnp.float32),
        scratch_types=[
            pltpu.VMEM((32, 128), jnp.int32),
            pltpu.VMEM((T,), jnp.int32),
            pltpu.VMEM((128, 2 * HS), jnp.float32),
            pltpu.VMEM((128, 2 * HS), jnp.float32),
            pltpu.SemaphoreType.DMA,
            pltpu.SemaphoreType.DMA,
        ],
    )
    def k(rk, osrc, odst, rank_vm, idx_vm, rows_a, rows_b, sem_a, sem_b):
        wid = lax.axis_index("s") * 2 + lax.axis_index("c")
        h = wid // 2
        a = wid % 2
        base = wid * T
        pltpu.sync_copy(rk.at[h, a], rank_vm)

        def mk_idx(i, _):
            for l in range(8):
                r16 = rank_vm[i, l * 16:(l + 1) * 16]
                idx_vm[pl.ds(i * 128 + l * 16, 16)] = r16 + base
            return 0

        lax.fori_loop(0, 32, mk_idx, 0)

        def g_step(i, _):
            j = i * 2
            cpa = pltpu.async_copy(
                osrc.at[idx_vm.at[pl.ds(j * 128, 128)]], rows_a, sem_a)
            cpb = pltpu.async_copy(
                osrc.at[idx_vm.at[pl.ds(j * 128 + 128, 128)]], rows_b, sem_b)
            cpa.wait()
            pltpu.sync_copy(rows_a, odst.at[pl.ds(base + j * 128, 128)])
            cpb.wait()
            pltpu.sync_copy(rows_b, odst.at[pl.ds(base + j * 128 + 128, 128)])
            return 0

        lax.fori_loop(0, T // 256, g_step, 0)

    return k(rank_k, outs)


def _k6_body(x00, x01, x10, x11, f_ref):
    halves = []
    for (xa, xb) in ((x00, x01), (x10, x11)):
        va = xa[...]
        vb = xb[...]
        l0 = va[:, 0:1]
        l1 = vb[:, 0:1]
        m = jnp.maximum(l0, l1)
        e0 = jnp.exp(l0 - m)
        e1 = jnp.exp(l1 - m)
        o = (va * e0 + vb * e1) / (e0 + e1)
        halves.append(o[:, HS:])
    f_ref[0] = jnp.concatenate(halves, axis=1)


def _combine(out_u):
    spec = lambda r: pl.BlockSpec((T, 2 * HS), lambda g, _r=r: (4 * g + _r, 0))
    return pl.pallas_call(
        _k6_body,
        grid=(NH // 2,),
        in_specs=[spec(0), spec(1), spec(2), spec(3)],
        out_specs=pl.BlockSpec((1, T, 2 * HS), lambda g: (0, 0, g)),
        out_shape=jax.ShapeDtypeStruct((1, T, HIDDEN), jnp.float32),
    )(out_u, out_u, out_u, out_u)


def kernel(decoder_states, hidden_states, W_qk, W_v):
    ds = decoder_states[0]
    hs = hidden_states[0]
    wq = W_qk.reshape(HIDDEN, NH, HS).transpose(1, 0, 2)
    wv = W_v.reshape(HIDDEN, NH, HS).transpose(1, 0, 2)
    wkv = jnp.concatenate([wq, wv], axis=-1)
    qtab = _project(ds, wq, _k1q_body)
    kvtab = _project(hs, wkv, _k1kv_body)

    rot = jax.random.normal(jax.random.key(HASH_SEED),
                            (NH, HS, NHASH, NB // 2), jnp.float32)
    rot4 = rot.transpose(0, 2, 1, 3)
    rot4 = jnp.concatenate([rot4, jnp.zeros_like(rot4)], axis=2)
    rank_q = _buckets_ranks(qtab, rot4)
    rank_k = _buckets_ranks(kvtab, rot4)

    poskl, qs, kvs = _sc_sort_gather(rank_q, rank_k, qtab, kvtab)
    out_s = _attention(qs, kvs, poskl)
    out_u = _sc_unsort(rank_k, out_s)
    return _combine(out_u)


# speedup vs baseline: 1.9155x; 1.1593x over previous
import functools

import jax
import jax.numpy as jnp
from jax import lax
from jax.experimental import pallas as pl
from jax.experimental.pallas import tpu as pltpu
from jax.experimental.pallas import tpu_sc as plsc

T = 4096
HIDDEN = 1024
NH = 16
HS = 64
CHUNK = 64
NHASH = 2
NB = 128
HASH_SEED = 1234
MASKVAL = -1e5
S = NHASH * T
NCH = S // CHUNK
R = NH * S


def _k1q_body(x_ref, w_ref, o_ref):
    y = jnp.dot(x_ref[...], w_ref[0], preferred_element_type=jnp.float32)
    i = pl.program_id(0)
    pos = (lax.broadcasted_iota(jnp.int32, (1024, 1), 0)
           + i * 1024).astype(jnp.float32)
    o_ref[...] = jnp.concatenate(
        [y, pos, jnp.zeros((1024, HS - 1), jnp.float32)], axis=1)


def _k1kv_body(x_ref, w_ref, o_ref):
    o_ref[...] = jnp.dot(x_ref[...], w_ref[0],
                         preferred_element_type=jnp.float32)


def _project(x, w_heads, body):
    W = w_heads.shape[-1]
    return pl.pallas_call(
        body,
        grid=(4, NH),
        in_specs=[
            pl.BlockSpec((1024, HIDDEN), lambda i, h: (i, 0)),
            pl.BlockSpec((1, HIDDEN, W), lambda i, h: (h, 0, 0)),
        ],
        out_specs=pl.BlockSpec((1024, 2 * HS), lambda i, h: (h * 4 + i, 0)),
        out_shape=jax.ShapeDtypeStruct((NH * T, 2 * HS), jnp.float32),
    )(x, w_heads)


def _k2_body(x_ref, rot_ref, rank_ref, ob_scr):
    x = x_ref[...]
    rot = rot_ref[0, 0]
    r = jnp.dot(x, rot, preferred_element_type=jnp.float32)

    ii = lax.broadcasted_iota(jnp.int32, (T, HS), 1).astype(jnp.float32)
    mx = jnp.max(r, axis=1, keepdims=True)
    mn = jnp.min(r, axis=1, keepdims=True)
    a1 = jnp.min(jnp.where(r == mx, ii, 64.0), axis=1, keepdims=True)
    a2 = jnp.min(jnp.where(r == mn, ii, 64.0), axis=1, keepdims=True) + 64.0
    bucket = jnp.where(mx >= -mn, a1, a2)

    bi = lax.broadcasted_iota(jnp.int32, (128, NB), 1).astype(jnp.float32)
    rows = lax.broadcasted_iota(jnp.int32, (128, 128), 0)
    cols = lax.broadcasted_iota(jnp.int32, (128, 128), 1)
    lstrict = (rows > cols).astype(jnp.float32)
    ustrict = (rows < cols).astype(jnp.float32)

    ones_row = jnp.ones((1, 128), jnp.float32)
    run = jnp.zeros((1, NB), jnp.float32)
    bases = []
    for blk in range(32):
        ob = (bucket[blk * 128:(blk + 1) * 128] == bi).astype(jnp.float32)
        ob_scr[blk * 128:(blk + 1) * 128, :] = ob
        bases.append(run)
        run = run + jnp.dot(ones_row, ob, preferred_element_type=jnp.float32)
    binbase = jnp.dot(run, ustrict, preferred_element_type=jnp.float32)

    for blk in range(32):
        ob = ob_scr[blk * 128:(blk + 1) * 128, :]
        pb = jnp.dot(lstrict, ob, preferred_element_type=jnp.float32)
        rk = jnp.sum(ob * (pb + binbase + bases[blk]), axis=1, keepdims=True)
        rank_ref[0, 0, blk:blk + 1, :] = \
            lax.transpose(rk, (1, 0)).astype(jnp.int32)


def _buckets_ranks(tab, rot4):
    return pl.pallas_call(
        _k2_body,
        grid=(NH, NHASH),
        in_specs=[
            pl.BlockSpec((T, 2 * HS), lambda h, a: (h, 0)),
            pl.BlockSpec((1, 1, 2 * HS, HS), lambda h, a: (h, a, 0, 0)),
        ],
        out_specs=pl.BlockSpec((1, 1, 32, 128), lambda h, a: (h, a, 0, 0)),
        out_shape=jax.ShapeDtypeStruct((NH, NHASH, 32, 128), jnp.int32),
        scratch_shapes=[pltpu.VMEM((T, 128), jnp.float32)],
    )(tab, rot4)


def _sc_mesh():
    return plsc.VectorSubcoreMesh(core_axis_name="c", subcore_axis_name="s")
_SC_PARAMS = pltpu.CompilerParams(needs_layout_passes=False,
                                  use_tc_tiling_on_sc=False)
_Z16 = lambda: jnp.zeros((16,), jnp.int32)


def _sc_sort_gather(rank_q, rank_k, qtab, kvtab):
    @functools.partial(
        pl.kernel,
        mesh=_sc_mesh(),
        compiler_params=_SC_PARAMS,
        out_type=[
            jax.ShapeDtypeStruct((NH, NCH, CHUNK), jnp.float32),
            jax.ShapeDtypeStruct((R, 2 * HS), jnp.float32),
            jax.ShapeDtypeStruct((R, 2 * HS), jnp.float32),
        ],
        scratch_types=[
            pltpu.VMEM((32, 128), jnp.int32),
            pltpu.VMEM((T,), jnp.int32),
            pltpu.VMEM((T,), jnp.int32),
            pltpu.VMEM((CHUNK, CHUNK), jnp.float32),
            pltpu.VMEM((128, 2 * HS), jnp.float32),
            pltpu.VMEM((128, 2 * HS), jnp.float32),
            pltpu.SemaphoreType.DMA,
            pltpu.SemaphoreType.DMA,
        ],
    )
    def k(rq, rk, qt, kvt, poskl, qs, kvs,
          rank_vm, idxq_vm, idxk_vm, plane_vm, rows_a, rows_b, sem_a, sem_b):
        wid = lax.axis_index("s") * 2 + lax.axis_index("c")
        h = wid // 2
        a = wid % 2

        def inv_q(i, _):
            for l in range(8):
                seq = i * 128 + l * 16 + lax.iota(jnp.int32, 16)
                rk16 = rank_vm[i, l * 16:(l + 1) * 16]
                plsc.store_scatter(idxq_vm, [rk16], seq + h * T)
            return 0

        def inv_k(i, _):
            for l in range(8):
                seq = i * 128 + l * 16 + lax.iota(jnp.int32, 16)
                rk16 = rank_vm[i, l * 16:(l + 1) * 16]
                plsc.store_scatter(idxk_vm, [rk16], seq + h * T)
                plsc.store_scatter(
                    plane_vm,
                    [lax.shift_right_logical(rk16, 6),
                     lax.bitwise_and(rk16, 63)],
                    seq.astype(jnp.float32))
            return 0

        def gather(tab, idx_vm, dst):
            def g_step(i, _):
                j = i * 2
                cpa = pltpu.async_copy(
                    tab.at[idx_vm.at[pl.ds(j * 128, 128)]], rows_a, sem_a)
                cpb = pltpu.async_copy(
                    tab.at[idx_vm.at[pl.ds(j * 128 + 128, 128)]],
                    rows_b, sem_b)
                cpa.wait()
                pltpu.sync_copy(rows_a,
                                dst.at[pl.ds(wid * T + j * 128, 128)])
                cpb.wait()
                pltpu.sync_copy(rows_b,
                                dst.at[pl.ds(wid * T + j * 128 + 128, 128)])
                return 0

            lax.fori_loop(0, T // 256, g_step, 0)

        pltpu.sync_copy(rq.at[h, a], rank_vm)
        lax.fori_loop(0, 32, inv_q, 0)
        pltpu.sync_copy(rk.at[h, a], rank_vm)
        lax.fori_loop(0, 32, inv_k, 0)
        pltpu.sync_copy(plane_vm, poskl.at[h, pl.ds(a * CHUNK, CHUNK)])
        gather(qt, idxq_vm, qs)
        gather(kvt, idxk_vm, kvs)

    return k(rank_q, rank_k, qtab, kvtab)


def _k4_body(qs_ref, kvs_ref, pkl_ref, kvw_ref, pkw_ref,
             out_ref, pk_scr, pp_scr):
    g = pl.program_id(1)
    lane = lax.broadcasted_iota(jnp.int32, (64, 2 * HS), 1)
    kmask = lane < HS
    kmaskf = kmask.astype(jnp.float32)
    eselm = (lane == HS).astype(jnp.float32)

    def norm_kv(rows):
        kk = rows * kmaskf
        var = jnp.sum(kk * kk, axis=1, keepdims=True) * (1.0 / HS)
        scale = lax.rsqrt(var + 1e-6) * (HS ** -0.5)
        return rows * jnp.where(kmask, scale, 1.0)

    @pl.when(g == 0)
    def _init():
        pk_scr[...] = norm_kv(kvw_ref[...])
        pp_scr[...] = pkw_ref[0, 15:16, :]

    kvprev = pk_scr[...]
    pprev = pp_scr[...]
    nt = (((1,), (1,)), ((), ()))
    for j in range(16):
        qc = qs_ref[j * 64:(j + 1) * 64, :]
        qm = qc * kmaskf
        kvn = norm_kv(kvs_ref[j * 64:(j + 1) * 64, :])
        pkc = pkl_ref[0, j:j + 1, :]
        pqc = jnp.sum(qc * eselm, axis=1, keepdims=True)
        d0 = lax.dot_general(qm, kvprev, nt,
                             preferred_element_type=jnp.float32)
        d1 = lax.dot_general(qm, kvn, nt,
                             preferred_element_type=jnp.float32)
        d0 = jnp.where(pqc != pprev, d0, MASKVAL)
        d1 = jnp.where(pqc != pkc, d1, MASKVAL)
        e0 = jnp.exp(d0)
        e1 = jnp.exp(d1)
        s = jnp.sum(e0, axis=1, keepdims=True) + \
            jnp.sum(e1, axis=1, keepdims=True)
        inv = 1.0 / s
        o = (jnp.dot(e0, kvprev, preferred_element_type=jnp.float32) +
             jnp.dot(e1, kvn, preferred_element_type=jnp.float32)) * inv
        l = jnp.log(s)
        out_ref[j * 64:(j + 1) * 64, :] = jnp.where(kmask, l, o)
        kvprev, pprev = kvn, pkc
    pk_scr[...] = kvprev
    pp_scr[...] = pprev


def _attention(qs, kvs, pkl):
    G = S // 8
    return pl.pallas_call(
        _k4_body,
        grid=(NH, 8),
        in_specs=[
            pl.BlockSpec((G, 2 * HS), lambda h, g: (h * 8 + g, 0)),
            pl.BlockSpec((G, 2 * HS), lambda h, g: (h * 8 + g, 0)),
            pl.BlockSpec((1, 16, CHUNK), lambda h, g: (h, g, 0)),
            pl.BlockSpec((64, 2 * HS), lambda h, g: (h * NCH + NCH - 1, 0)),
            pl.BlockSpec((1, 16, CHUNK), lambda h, g: (h, 7, 0)),
        ],
        out_specs=pl.BlockSpec((G, 2 * HS), lambda h, g: (h * 8 + g, 0)),
        out_shape=jax.ShapeDtypeStruct((R, 2 * HS), jnp.float32),
        scratch_shapes=[
            pltpu.VMEM((64, 2 * HS), jnp.float32),
            pltpu.VMEM((1, HS), jnp.float32),
        ],
    )(qs, kvs, pkl, kvs, pkl)


def _sc_unsort(rank_k, outs):
    @functools.partial(
        pl.kernel,
        mesh=_sc_mesh(),
        compiler_params=_SC_PARAMS,
        out_type=jax.ShapeDtypeStruct((R, 2 * HS), jnp.float32),
        scratch_types=[
            pltpu.VMEM((32, 128), jnp.int32),
            pltpu.VMEM((T,), jnp.int32),
            pltpu.VMEM((128, 2 * HS), jnp.float32),
            pltpu.VMEM((128, 2 * HS), jnp.float32),
            pltpu.SemaphoreType.DMA,
            pltpu.SemaphoreType.DMA,
        ],
    )
    def k(rk, osrc, odst, rank_vm, idx_vm, rows_a, rows_b, sem_a, sem_b):
        wid = lax.axis_index("s") * 2 + lax.axis_index("c")
        h = wid // 2
        a = wid % 2
        base = wid * T
        pltpu.sync_copy(rk.at[h, a], rank_vm)

        def mk_idx(i, _):
            for l in range(8):
                r16 = rank_vm[i, l * 16:(l + 1) * 16]
                idx_vm[pl.ds(i * 128 + l * 16, 16)] = r16 + base
            return 0

        lax.fori_loop(0, 32, mk_idx, 0)

        def g_step(i, _):
            j = i * 2
            cpa = pltpu.async_copy(
                osrc.at[idx_vm.at[pl.ds(j * 128, 128)]], rows_a, sem_a)
            cpb = pltpu.async_copy(
                osrc.at[idx_vm.at[pl.ds(j * 128 + 128, 128)]], rows_b, sem_b)
            cpa.wait()
            pltpu.sync_copy(rows_a, odst.at[pl.ds(base + j * 128, 128)])
            cpb.wait()
            pltpu.sync_copy(rows_b, odst.at[pl.ds(base + j * 128 + 128, 128)])
            return 0

        lax.fori_loop(0, T // 256, g_step, 0)

    return k(rank_k, outs)


def _k6_body(x00, x01, x10, x11, f_ref):
    halves = []
    for (xa, xb) in ((x00, x01), (x10, x11)):
        va = xa[...]
        vb = xb[...]
        l0 = va[:, 0:1]
        l1 = vb[:, 0:1]
        m = jnp.maximum(l0, l1)
        e0 = jnp.exp(l0 - m)
        e1 = jnp.exp(l1 - m)
        o = (va * e0 + vb * e1) / (e0 + e1)
        halves.append(o[:, HS:])
    f_ref[0] = jnp.concatenate(halves, axis=1)


def _combine(out_u):
    spec = lambda r: pl.BlockSpec((T, 2 * HS), lambda g, _r=r: (4 * g + _r, 0))
    return pl.pallas_call(
        _k6_body,
        grid=(NH // 2,),
        in_specs=[spec(0), spec(1), spec(2), spec(3)],
        out_specs=pl.BlockSpec((1, T, 2 * HS), lambda g: (0, 0, g)),
        out_shape=jax.ShapeDtypeStruct((1, T, HIDDEN), jnp.float32),
    )(out_u, out_u, out_u, out_u)


def kernel(decoder_states, hidden_states, W_qk, W_v):
    ds = decoder_states[0]
    hs = hidden_states[0]
    wq = W_qk.reshape(HIDDEN, NH, HS).transpose(1, 0, 2)
    wv = W_v.reshape(HIDDEN, NH, HS).transpose(1, 0, 2)
    wkv = jnp.concatenate([wq, wv], axis=-1)
    qtab = _project(ds, wq, _k1q_body)
    kvtab = _project(hs, wkv, _k1kv_body)

    rot = jax.random.normal(jax.random.key(HASH_SEED),
                            (NH, HS, NHASH, NB // 2), jnp.float32)
    rot4 = rot.transpose(0, 2, 1, 3)
    rot4 = jnp.concatenate([rot4, jnp.zeros_like(rot4)], axis=2)
    rank_q = _buckets_ranks(qtab, rot4)
    rank_k = _buckets_ranks(kvtab, rot4)

    poskl, qs, kvs = _sc_sort_gather(rank_q, rank_k, qtab, kvtab)
    out_s = _attention(qs, kvs, poskl)
    out_u = _sc_unsort(rank_k, out_s)
    return _combine(out_u)


# speedup vs baseline: 1.9916x; 1.0397x over previous
import functools

import jax
import jax.numpy as jnp
from jax import lax
from jax.experimental import pallas as pl
from jax.experimental.pallas import tpu as pltpu
from jax.experimental.pallas import tpu_sc as plsc

T = 4096
HIDDEN = 1024
NH = 16
HS = 64
CHUNK = 64
NHASH = 2
NB = 128
HASH_SEED = 1234
MASKVAL = -1e5
S = NHASH * T
NCH = S // CHUNK
R = NH * S


def _k1q_body(x_ref, w_ref, o_ref):
    y = jnp.dot(x_ref[...], w_ref[0], preferred_element_type=jnp.float32)
    i = pl.program_id(0)
    pos = (lax.broadcasted_iota(jnp.int32, (1024, 1), 0)
           + i * 1024).astype(jnp.float32)
    o_ref[...] = jnp.concatenate(
        [y, pos, jnp.zeros((1024, HS - 1), jnp.float32)], axis=1)


def _k1kv_body(x_ref, w_ref, o_ref):
    o_ref[...] = jnp.dot(x_ref[...], w_ref[0],
                         preferred_element_type=jnp.float32)


def _project(x, w_heads, body):
    W = w_heads.shape[-1]
    return pl.pallas_call(
        body,
        grid=(4, NH),
        in_specs=[
            pl.BlockSpec((1024, HIDDEN), lambda i, h: (i, 0)),
            pl.BlockSpec((1, HIDDEN, W), lambda i, h: (h, 0, 0)),
        ],
        out_specs=pl.BlockSpec((1024, 2 * HS), lambda i, h: (h * 4 + i, 0)),
        out_shape=jax.ShapeDtypeStruct((NH * T, 2 * HS), jnp.float32),
    )(x, w_heads)


def _k2_body(x_ref, rot_ref, rank_ref, ob_scr):
    x = x_ref[...]
    rot = rot_ref[0, 0]
    r = jnp.dot(x, rot, preferred_element_type=jnp.float32)

    ii = lax.broadcasted_iota(jnp.int32, (T, HS), 1).astype(jnp.float32)
    mx = jnp.max(r, axis=1, keepdims=True)
    mn = jnp.min(r, axis=1, keepdims=True)
    a1 = jnp.min(jnp.where(r == mx, ii, 64.0), axis=1, keepdims=True)
    a2 = jnp.min(jnp.where(r == mn, ii, 64.0), axis=1, keepdims=True) + 64.0
    bucket = jnp.where(mx >= -mn, a1, a2)

    bi = lax.broadcasted_iota(jnp.int32, (128, NB), 1).astype(jnp.float32)
    rows = lax.broadcasted_iota(jnp.int32, (128, 128), 0)
    cols = lax.broadcasted_iota(jnp.int32, (128, 128), 1)
    lstrict = (rows > cols).astype(jnp.float32)
    ustrict = (rows < cols).astype(jnp.float32)

    ones_row = jnp.ones((1, 128), jnp.float32)
    run = jnp.zeros((1, NB), jnp.float32)
    bases = []
    for blk in range(32):
        ob = (bucket[blk * 128:(blk + 1) * 128] == bi).astype(jnp.float32)
        ob_scr[blk * 128:(blk + 1) * 128, :] = ob
        bases.append(run)
        run = run + jnp.dot(ones_row, ob, preferred_element_type=jnp.float32)
    binbase = jnp.dot(run, ustrict, preferred_element_type=jnp.float32)

    for blk in range(32):
        ob = ob_scr[blk * 128:(blk + 1) * 128, :]
        pb = jnp.dot(lstrict, ob, preferred_element_type=jnp.float32)
        rk = jnp.sum(ob * (pb + binbase + bases[blk]), axis=1, keepdims=True)
        rank_ref[0, 0, blk:blk + 1, :] = \
            lax.transpose(rk, (1, 0)).astype(jnp.int32)


def _buckets_ranks(tab, rot4):
    return pl.pallas_call(
        _k2_body,
        grid=(NH, NHASH),
        in_specs=[
            pl.BlockSpec((T, 2 * HS), lambda h, a: (h, 0)),
            pl.BlockSpec((1, 1, 2 * HS, HS), lambda h, a: (h, a, 0, 0)),
        ],
        out_specs=pl.BlockSpec((1, 1, 32, 128), lambda h, a: (h, a, 0, 0)),
        out_shape=jax.ShapeDtypeStruct((NH, NHASH, 32, 128), jnp.int32),
        scratch_shapes=[pltpu.VMEM((T, 128), jnp.float32)],
    )(tab, rot4)


def _sc_mesh():
    return plsc.VectorSubcoreMesh(core_axis_name="c", subcore_axis_name="s")
_SC_PARAMS = pltpu.CompilerParams(needs_layout_passes=False,
                                  use_tc_tiling_on_sc=False)
_Z16 = lambda: jnp.zeros((16,), jnp.int32)


def _sc_gather_q(rank_q, qtab):
    @functools.partial(
        pl.kernel,
        mesh=_sc_mesh(),
        compiler_params=_SC_PARAMS,
        out_type=jax.ShapeDtypeStruct((R, 2 * HS), jnp.float32),
        scratch_types=[
            pltpu.VMEM((32, 128), jnp.int32),
            pltpu.VMEM((T,), jnp.int32),
            pltpu.VMEM((128, 2 * HS), jnp.float32),
            pltpu.VMEM((128, 2 * HS), jnp.float32),
            pltpu.SemaphoreType.DMA,
            pltpu.SemaphoreType.DMA,
        ],
    )
    def k(rq, qt, qs, rank_vm, idx_vm, rows_a, rows_b, sem_a, sem_b):
        wid = lax.axis_index("s") * 2 + lax.axis_index("c")
        h = wid // 2
        a = wid % 2

        def inv_q(i, _):
            for l in range(8):
                seq = i * 128 + l * 16 + lax.iota(jnp.int32, 16)
                rk16 = rank_vm[i, l * 16:(l + 1) * 16]
                plsc.store_scatter(idx_vm, [rk16], seq + h * T)
            return 0

        def g_step(i, _):
            j = i * 2
            cpa = pltpu.async_copy(
                qt.at[idx_vm.at[pl.ds(j * 128, 128)]], rows_a, sem_a)
            cpb = pltpu.async_copy(
                qt.at[idx_vm.at[pl.ds(j * 128 + 128, 128)]], rows_b, sem_b)
            cpa.wait()
            pltpu.sync_copy(rows_a, qs.at[pl.ds(wid * T + j * 128, 128)])
            cpb.wait()
            pltpu.sync_copy(rows_b,
                            qs.at[pl.ds(wid * T + j * 128 + 128, 128)])
            return 0

        pltpu.sync_copy(rq.at[h, a], rank_vm)
        lax.fori_loop(0, 32, inv_q, 0)
        lax.fori_loop(0, T // 256, g_step, 0)

    return k(rank_q, qtab)


def _sc_gather_kv(rank_k, kvtab):
    @functools.partial(
        pl.kernel,
        mesh=_sc_mesh(),
        compiler_params=_SC_PARAMS,
        out_type=[
            jax.ShapeDtypeStruct((NH, NCH, CHUNK), jnp.float32),
            jax.ShapeDtypeStruct((R, 2 * HS), jnp.float32),
        ],
        scratch_types=[
            pltpu.VMEM((32, 128), jnp.int32),
            pltpu.VMEM((T,), jnp.int32),
            pltpu.VMEM((CHUNK, CHUNK), jnp.float32),
            pltpu.VMEM((128, 2 * HS), jnp.float32),
            pltpu.VMEM((128, 2 * HS), jnp.float32),
            pltpu.SemaphoreType.DMA,
            pltpu.SemaphoreType.DMA,
        ],
    )
    def k(rk, kvt, poskl, kvs,
          rank_vm, idx_vm, plane_vm, rows_a, rows_b, sem_a, sem_b):
        wid = lax.axis_index("s") * 2 + lax.axis_index("c")
        h = wid // 2
        a = wid % 2

        def inv_k(i, _):
            for l in range(8):
                seq = i * 128 + l * 16 + lax.iota(jnp.int32, 16)
                rk16 = rank_vm[i, l * 16:(l + 1) * 16]
                plsc.store_scatter(idx_vm, [rk16], seq + h * T)
                plsc.store_scatter(
                    plane_vm,
                    [lax.shift_right_logical(rk16, 6),
                     lax.bitwise_and(rk16, 63)],
                    seq.astype(jnp.float32))
            return 0

        def g_step(i, _):
            j = i * 2
            cpa = pltpu.async_copy(
                kvt.at[idx_vm.at[pl.ds(j * 128, 128)]], rows_a, sem_a)
            cpb = pltpu.async_copy(
                kvt.at[idx_vm.at[pl.ds(j * 128 + 128, 128)]], rows_b, sem_b)
            cpa.wait()
            pltpu.sync_copy(rows_a, kvs.at[pl.ds(wid * T + j * 128, 128)])
            cpb.wait()
            pltpu.sync_copy(rows_b,
                            kvs.at[pl.ds(wid * T + j * 128 + 128, 128)])
            return 0

        pltpu.sync_copy(rk.at[h, a], rank_vm)
        lax.fori_loop(0, 32, inv_k, 0)
        pltpu.sync_copy(plane_vm, poskl.at[h, pl.ds(a * CHUNK, CHUNK)])
        lax.fori_loop(0, T // 256, g_step, 0)

    return k(rank_k, kvtab)


def _k4_body(qs_ref, kvs_ref, pkl_ref, kvw_ref, pkw_ref,
             out_ref, pk_scr, pp_scr):
    g = pl.program_id(1)
    lane = lax.broadcasted_iota(jnp.int32, (64, 2 * HS), 1)
    kmask = lane < HS
    kmaskf = kmask.astype(jnp.float32)
    eselm = (lane == HS).astype(jnp.float32)

    def norm_kv(rows):
        kk = rows * kmaskf
        var = jnp.sum(kk * kk, axis=1, keepdims=True) * (1.0 / HS)
        scale = lax.rsqrt(var + 1e-6) * (HS ** -0.5)
        return rows * jnp.where(kmask, scale, 1.0)

    @pl.when(g == 0)
    def _init():
        pk_scr[...] = norm_kv(kvw_ref[...])
        pp_scr[...] = pkw_ref[0, 15:16, :]

    kvprev = pk_scr[...]
    pprev = pp_scr[...]
    nt = (((1,), (1,)), ((), ()))
    for j in range(16):
        qc = qs_ref[j * 64:(j + 1) * 64, :]
        qm = qc * kmaskf
        kvn = norm_kv(kvs_ref[j * 64:(j + 1) * 64, :])
        pkc = pkl_ref[0, j:j + 1, :]
        pqc = jnp.sum(qc * eselm, axis=1, keepdims=True)
        d0 = lax.dot_general(qm, kvprev, nt,
                             preferred_element_type=jnp.float32)
        d1 = lax.dot_general(qm, kvn, nt,
                             preferred_element_type=jnp.float32)
        d0 = jnp.where(pqc != pprev, d0, MASKVAL)
        d1 = jnp.where(pqc != pkc, d1, MASKVAL)
        e0 = jnp.exp(d0)
        e1 = jnp.exp(d1)
        s = jnp.sum(e0, axis=1, keepdims=True) + \
            jnp.sum(e1, axis=1, keepdims=True)
        inv = 1.0 / s
        o = (jnp.dot(e0, kvprev, preferred_element_type=jnp.float32) +
             jnp.dot(e1, kvn, preferred_element_type=jnp.float32)) * inv
        l = jnp.log(s)
        out_ref[j * 64:(j + 1) * 64, :] = jnp.where(kmask, l, o)
        kvprev, pprev = kvn, pkc
    pk_scr[...] = kvprev
    pp_scr[...] = pprev


def _attention(qs, kvs, pkl):
    G = S // 8
    return pl.pallas_call(
        _k4_body,
        grid=(NH, 8),
        in_specs=[
            pl.BlockSpec((G, 2 * HS), lambda h, g: (h * 8 + g, 0)),
            pl.BlockSpec((G, 2 * HS), lambda h, g: (h * 8 + g, 0)),
            pl.BlockSpec((1, 16, CHUNK), lambda h, g: (h, g, 0)),
            pl.BlockSpec((64, 2 * HS), lambda h, g: (h * NCH + NCH - 1, 0)),
            pl.BlockSpec((1, 16, CHUNK), lambda h, g: (h, 7, 0)),
        ],
        out_specs=pl.BlockSpec((G, 2 * HS), lambda h, g: (h * 8 + g, 0)),
        out_shape=jax.ShapeDtypeStruct((R, 2 * HS), jnp.float32),
        scratch_shapes=[
            pltpu.VMEM((64, 2 * HS), jnp.float32),
            pltpu.VMEM((1, HS), jnp.float32),
        ],
    )(qs, kvs, pkl, kvs, pkl)


def _sc_unsort(rank_k, outs):
    @functools.partial(
        pl.kernel,
        mesh=_sc_mesh(),
        compiler_params=_SC_PARAMS,
        out_type=jax.ShapeDtypeStruct((R, 2 * HS), jnp.float32),
        scratch_types=[
            pltpu.VMEM((32, 128), jnp.int32),
            pltpu.VMEM((T,), jnp.int32),
            pltpu.VMEM((128, 2 * HS), jnp.float32),
            pltpu.VMEM((128, 2 * HS), jnp.float32),
            pltpu.SemaphoreType.DMA,
            pltpu.SemaphoreType.DMA,
        ],
    )
    def k(rk, osrc, odst, rank_vm, idx_vm, rows_a, rows_b, sem_a, sem_b):
        wid = lax.axis_index("s") * 2 + lax.axis_index("c")
        h = wid // 2
        a = wid % 2
        base = wid * T
        pltpu.sync_copy(rk.at[h, a], rank_vm)

        def mk_idx(i, _):
            for l in range(8):
                r16 = rank_vm[i, l * 16:(l + 1) * 16]
                idx_vm[pl.ds(i * 128 + l * 16, 16)] = r16 + base
            return 0

        lax.fori_loop(0, 32, mk_idx, 0)

        def g_step(i, _):
            j = i * 2
            cpa = pltpu.async_copy(
                osrc.at[idx_vm.at[pl.ds(j * 128, 128)]], rows_a, sem_a)
            cpb = pltpu.async_copy(
                osrc.at[idx_vm.at[pl.ds(j * 128 + 128, 128)]], rows_b, sem_b)
            cpa.wait()
            pltpu.sync_copy(rows_a, odst.at[pl.ds(base + j * 128, 128)])
            cpb.wait()
            pltpu.sync_copy(rows_b, odst.at[pl.ds(base + j * 128 + 128, 128)])
            return 0

        lax.fori_loop(0, T // 256, g_step, 0)

    return k(rank_k, outs)


def _k6_body(x00, x01, x10, x11, f_ref):
    halves = []
    for (xa, xb) in ((x00, x01), (x10, x11)):
        va = xa[...]
        vb = xb[...]
        l0 = va[:, 0:1]
        l1 = vb[:, 0:1]
        m = jnp.maximum(l0, l1)
        e0 = jnp.exp(l0 - m)
        e1 = jnp.exp(l1 - m)
        o = (va * e0 + vb * e1) / (e0 + e1)
        halves.append(o[:, HS:])
    f_ref[0] = jnp.concatenate(halves, axis=1)


def _combine(out_u):
    spec = lambda r: pl.BlockSpec((T, 2 * HS), lambda g, _r=r: (4 * g + _r, 0))
    return pl.pallas_call(
        _k6_body,
        grid=(NH // 2,),
        in_specs=[spec(0), spec(1), spec(2), spec(3)],
        out_specs=pl.BlockSpec((1, T, 2 * HS), lambda g: (0, 0, g)),
        out_shape=jax.ShapeDtypeStruct((1, T, HIDDEN), jnp.float32),
    )(out_u, out_u, out_u, out_u)


def kernel(decoder_states, hidden_states, W_qk, W_v):
    ds = decoder_states[0]
    hs = hidden_states[0]
    wq = W_qk.reshape(HIDDEN, NH, HS).transpose(1, 0, 2)
    wv = W_v.reshape(HIDDEN, NH, HS).transpose(1, 0, 2)
    wkv = jnp.concatenate([wq, wv], axis=-1)
    qtab = _project(ds, wq, _k1q_body)

    rot = jax.random.normal(jax.random.key(HASH_SEED),
                            (NH, HS, NHASH, NB // 2), jnp.float32)
    rot4 = rot.transpose(0, 2, 1, 3)
    rot4 = jnp.concatenate([rot4, jnp.zeros_like(rot4)], axis=2)
    rank_q = _buckets_ranks(qtab, rot4)
    qs = _sc_gather_q(rank_q, qtab)

    kvtab = _project(hs, wkv, _k1kv_body)
    rank_k = _buckets_ranks(kvtab, rot4)
    poskl, kvs = _sc_gather_kv(rank_k, kvtab)
    out_s = _attention(qs, kvs, poskl)
    out_u = _sc_unsort(rank_k, out_s)
    return _combine(out_u)


# speedup vs baseline: 2.1273x; 1.0681x over previous
import functools

import jax
import jax.numpy as jnp
from jax import lax
from jax.experimental import pallas as pl
from jax.experimental.pallas import tpu as pltpu
from jax.experimental.pallas import tpu_sc as plsc

T = 4096
HIDDEN = 1024
NH = 16
HS = 64
CHUNK = 64
NHASH = 2
NB = 128
HASH_SEED = 1234
MASKVAL = -1e5
S = NHASH * T
NCH = S // CHUNK
R = NH * S


def _k1q_body(x_ref, w_ref, o_ref):
    x = x_ref[...]
    i = pl.program_id(0)
    pos = (lax.broadcasted_iota(jnp.int32, (1024, 1), 0)
           + i * 1024).astype(jnp.float32)
    pad = jnp.zeros((1024, HS - 1), jnp.float32)
    for h in range(NH):
        y = jnp.dot(x, w_ref[h], preferred_element_type=jnp.float32)
        o_ref[h] = jnp.concatenate([y, pos, pad], axis=1)


def _k1kv_body(x_ref, w_ref, o_ref):
    x = x_ref[...]
    for h in range(NH):
        o_ref[h] = jnp.dot(x, w_ref[h], preferred_element_type=jnp.float32)


def _project(x, w_heads, body):
    W = w_heads.shape[-1]
    return pl.pallas_call(
        body,
        grid=(4,),
        in_specs=[
            pl.BlockSpec((1024, HIDDEN), lambda i: (i, 0)),
            pl.BlockSpec((NH, HIDDEN, W), lambda i: (0, 0, 0)),
        ],
        out_specs=pl.BlockSpec((NH, 1024, 2 * HS), lambda i: (0, i, 0)),
        out_shape=jax.ShapeDtypeStruct((NH, T, 2 * HS), jnp.float32),
    )(x, w_heads).reshape(NH * T, 2 * HS)


def _k2_body(x_ref, rot_ref, rank_ref, ob_scr):
    x = x_ref[...]
    rot = rot_ref[0, 0]
    r = jnp.dot(x, rot, preferred_element_type=jnp.float32)

    ii = lax.broadcasted_iota(jnp.int32, (T, HS), 1).astype(jnp.float32)
    mx = jnp.max(r, axis=1, keepdims=True)
    mn = jnp.min(r, axis=1, keepdims=True)
    a1 = jnp.min(jnp.where(r == mx, ii, 64.0), axis=1, keepdims=True)
    a2 = jnp.min(jnp.where(r == mn, ii, 64.0), axis=1, keepdims=True) + 64.0
    bucket = jnp.where(mx >= -mn, a1, a2)

    bi = lax.broadcasted_iota(jnp.int32, (128, NB), 1).astype(jnp.float32)
    rows = lax.broadcasted_iota(jnp.int32, (128, 128), 0)
    cols = lax.broadcasted_iota(jnp.int32, (128, 128), 1)
    lstrict = (rows > cols).astype(jnp.float32)
    ustrict = (rows < cols).astype(jnp.float32)

    ones_row = jnp.ones((1, 128), jnp.float32)
    run = jnp.zeros((1, NB), jnp.float32)
    bases = []
    for blk in range(32):
        ob = (bucket[blk * 128:(blk + 1) * 128] == bi).astype(jnp.float32)
        ob_scr[blk * 128:(blk + 1) * 128, :] = ob
        bases.append(run)
        run = run + jnp.dot(ones_row, ob, preferred_element_type=jnp.float32)
    binbase = jnp.dot(run, ustrict, preferred_element_type=jnp.float32)

    for blk in range(32):
        ob = ob_scr[blk * 128:(blk + 1) * 128, :]
        pb = jnp.dot(lstrict, ob, preferred_element_type=jnp.float32)
        rk = jnp.sum(ob * (pb + binbase + bases[blk]), axis=1, keepdims=True)
        rank_ref[0, 0, blk:blk + 1, :] = \
            lax.transpose(rk, (1, 0)).astype(jnp.int32)


def _buckets_ranks(tab, rot4):
    return pl.pallas_call(
        _k2_body,
        grid=(NH, NHASH),
        in_specs=[
            pl.BlockSpec((T, 2 * HS), lambda h, a: (h, 0)),
            pl.BlockSpec((1, 1, 2 * HS, HS), lambda h, a: (h, a, 0, 0)),
        ],
        out_specs=pl.BlockSpec((1, 1, 32, 128), lambda h, a: (h, a, 0, 0)),
        out_shape=jax.ShapeDtypeStruct((NH, NHASH, 32, 128), jnp.int32),
        scratch_shapes=[pltpu.VMEM((T, 128), jnp.float32)],
    )(tab, rot4)


def _sc_mesh():
    return plsc.VectorSubcoreMesh(core_axis_name="c", subcore_axis_name="s")
_SC_PARAMS = pltpu.CompilerParams(needs_layout_passes=False,
                                  use_tc_tiling_on_sc=False)
_Z16 = lambda: jnp.zeros((16,), jnp.int32)


def _sc_gather_q(rank_q, qtab):
    @functools.partial(
        pl.kernel,
        mesh=_sc_mesh(),
        compiler_params=_SC_PARAMS,
        out_type=jax.ShapeDtypeStruct((R, 2 * HS), jnp.float32),
        scratch_types=[
            pltpu.VMEM((32, 128), jnp.int32),
            pltpu.VMEM((T,), jnp.int32),
            pltpu.VMEM((128, 2 * HS), jnp.float32),
            pltpu.VMEM((128, 2 * HS), jnp.float32),
            pltpu.SemaphoreType.DMA,
            pltpu.SemaphoreType.DMA,
        ],
    )
    def k(rq, qt, qs, rank_vm, idx_vm, rows_a, rows_b, sem_a, sem_b):
        wid = lax.axis_index("s") * 2 + lax.axis_index("c")
        h = wid // 2
        a = wid % 2

        def inv_q(i, _):
            for l in range(8):
                seq = i * 128 + l * 16 + lax.iota(jnp.int32, 16)
                rk16 = rank_vm[i, l * 16:(l + 1) * 16]
                plsc.store_scatter(idx_vm, [rk16], seq + h * T)
            return 0

        def g_step(i, _):
            j = i * 2
            cpa = pltpu.async_copy(
                qt.at[idx_vm.at[pl.ds(j * 128, 128)]], rows_a, sem_a)
            cpb = pltpu.async_copy(
                qt.at[idx_vm.at[pl.ds(j * 128 + 128, 128)]], rows_b, sem_b)
            cpa.wait()
            pltpu.sync_copy(rows_a, qs.at[pl.ds(wid * T + j * 128, 128)])
            cpb.wait()
            pltpu.sync_copy(rows_b,
                            qs.at[pl.ds(wid * T + j * 128 + 128, 128)])
            return 0

        pltpu.sync_copy(rq.at[h, a], rank_vm)
        lax.fori_loop(0, 32, inv_q, 0)
        lax.fori_loop(0, T // 256, g_step, 0)

    return k(rank_q, qtab)


def _sc_gather_kv(rank_k, kvtab):
    @functools.partial(
        pl.kernel,
        mesh=_sc_mesh(),
        compiler_params=_SC_PARAMS,
        out_type=[
            jax.ShapeDtypeStruct((NH, NCH, CHUNK), jnp.float32),
            jax.ShapeDtypeStruct((R, 2 * HS), jnp.float32),
        ],
        scratch_types=[
            pltpu.VMEM((32, 128), jnp.int32),
            pltpu.VMEM((T,), jnp.int32),
            pltpu.VMEM((CHUNK, CHUNK), jnp.float32),
            pltpu.VMEM((128, 2 * HS), jnp.float32),
            pltpu.VMEM((128, 2 * HS), jnp.float32),
            pltpu.SemaphoreType.DMA,
            pltpu.SemaphoreType.DMA,
        ],
    )
    def k(rk, kvt, poskl, kvs,
          rank_vm, idx_vm, plane_vm, rows_a, rows_b, sem_a, sem_b):
        wid = lax.axis_index("s") * 2 + lax.axis_index("c")
        h = wid // 2
        a = wid % 2

        def inv_k(i, _):
            for l in range(8):
                seq = i * 128 + l * 16 + lax.iota(jnp.int32, 16)
                rk16 = rank_vm[i, l * 16:(l + 1) * 16]
                plsc.store_scatter(idx_vm, [rk16], seq + h * T)
                plsc.store_scatter(
                    plane_vm,
                    [lax.shift_right_logical(rk16, 6),
                     lax.bitwise_and(rk16, 63)],
                    seq.astype(jnp.float32))
            return 0

        def g_step(i, _):
            j = i * 2
            cpa = pltpu.async_copy(
                kvt.at[idx_vm.at[pl.ds(j * 128, 128)]], rows_a, sem_a)
            cpb = pltpu.async_copy(
                kvt.at[idx_vm.at[pl.ds(j * 128 + 128, 128)]], rows_b, sem_b)
            cpa.wait()
            pltpu.sync_copy(rows_a, kvs.at[pl.ds(wid * T + j * 128, 128)])
            cpb.wait()
            pltpu.sync_copy(rows_b,
                            kvs.at[pl.ds(wid * T + j * 128 + 128, 128)])
            return 0

        pltpu.sync_copy(rk.at[h, a], rank_vm)
        lax.fori_loop(0, 32, inv_k, 0)
        pltpu.sync_copy(plane_vm, poskl.at[h, pl.ds(a * CHUNK, CHUNK)])
        lax.fori_loop(0, T // 256, g_step, 0)

    return k(rank_k, kvtab)


def _k4_body(qs_ref, kvs_ref, pkl_ref, kvw_ref, pkw_ref,
             out_ref, pk_scr, pp_scr):
    g = pl.program_id(1)
    lane = lax.broadcasted_iota(jnp.int32, (64, 2 * HS), 1)
    kmask = lane < HS
    kmaskf = kmask.astype(jnp.float32)
    eselm = (lane == HS).astype(jnp.float32)

    def norm_kv(rows):
        kk = rows * kmaskf
        var = jnp.sum(kk * kk, axis=1, keepdims=True) * (1.0 / HS)
        scale = lax.rsqrt(var + 1e-6) * (HS ** -0.5)
        return rows * jnp.where(kmask, scale, 1.0)

    @pl.when(g == 0)
    def _init():
        pk_scr[...] = norm_kv(kvw_ref[...])
        pp_scr[...] = pkw_ref[0, 15:16, :]

    kvprev = pk_scr[...]
    pprev = pp_scr[...]
    nt = (((1,), (1,)), ((), ()))
    for j in range(16):
        qc = qs_ref[j * 64:(j + 1) * 64, :]
        qm = qc * kmaskf
        kvn = norm_kv(kvs_ref[j * 64:(j + 1) * 64, :])
        pkc = pkl_ref[0, j:j + 1, :]
        pqc = jnp.sum(qc * eselm, axis=1, keepdims=True)
        d0 = lax.dot_general(qm, kvprev, nt,
                             preferred_element_type=jnp.float32)
        d1 = lax.dot_general(qm, kvn, nt,
                             preferred_element_type=jnp.float32)
        d0 = jnp.where(pqc != pprev, d0, MASKVAL)
        d1 = jnp.where(pqc != pkc, d1, MASKVAL)
        e0 = jnp.exp(d0)
        e1 = jnp.exp(d1)
        s = jnp.sum(e0, axis=1, keepdims=True) + \
            jnp.sum(e1, axis=1, keepdims=True)
        inv = 1.0 / s
        o = (jnp.dot(e0, kvprev, preferred_element_type=jnp.float32) +
             jnp.dot(e1, kvn, preferred_element_type=jnp.float32)) * inv
        l = jnp.log(s)
        out_ref[j * 64:(j + 1) * 64, :] = jnp.where(kmask, l, o)
        kvprev, pprev = kvn, pkc
    pk_scr[...] = kvprev
    pp_scr[...] = pprev


def _attention(qs, kvs, pkl):
    G = S // 8
    return pl.pallas_call(
        _k4_body,
        grid=(NH, 8),
        in_specs=[
            pl.BlockSpec((G, 2 * HS), lambda h, g: (h * 8 + g, 0)),
            pl.BlockSpec((G, 2 * HS), lambda h, g: (h * 8 + g, 0)),
            pl.BlockSpec((1, 16, CHUNK), lambda h, g: (h, g, 0)),
            pl.BlockSpec((64, 2 * HS), lambda h, g: (h * NCH + NCH - 1, 0)),
            pl.BlockSpec((1, 16, CHUNK), lambda h, g: (h, 7, 0)),
        ],
        out_specs=pl.BlockSpec((G, 2 * HS), lambda h, g: (h * 8 + g, 0)),
        out_shape=jax.ShapeDtypeStruct((R, 2 * HS), jnp.float32),
        scratch_shapes=[
            pltpu.VMEM((64, 2 * HS), jnp.float32),
            pltpu.VMEM((1, HS), jnp.float32),
        ],
    )(qs, kvs, pkl, kvs, pkl)


def _sc_unsort(rank_k, outs):
    @functools.partial(
        pl.kernel,
        mesh=_sc_mesh(),
        compiler_params=_SC_PARAMS,
        out_type=jax.ShapeDtypeStruct((R, 2 * HS), jnp.float32),
        scratch_types=[
            pltpu.VMEM((32, 128), jnp.int32),
            pltpu.VMEM((T,), jnp.int32),
            pltpu.VMEM((128, 2 * HS), jnp.float32),
            pltpu.VMEM((128, 2 * HS), jnp.float32),
            pltpu.SemaphoreType.DMA,
            pltpu.SemaphoreType.DMA,
        ],
    )
    def k(rk, osrc, odst, rank_vm, idx_vm, rows_a, rows_b, sem_a, sem_b):
        wid = lax.axis_index("s") * 2 + lax.axis_index("c")
        h = wid // 2
        a = wid % 2
        base = wid * T
        pltpu.sync_copy(rk.at[h, a], rank_vm)

        def mk_idx(i, _):
            for l in range(8):
                r16 = rank_vm[i, l * 16:(l + 1) * 16]
                idx_vm[pl.ds(i * 128 + l * 16, 16)] = r16 + base
            return 0

        lax.fori_loop(0, 32, mk_idx, 0)

        def g_step(i, _):
            j = i * 2
            cpa = pltpu.async_copy(
                osrc.at[idx_vm.at[pl.ds(j * 128, 128)]], rows_a, sem_a)
            cpb = pltpu.async_copy(
                osrc.at[idx_vm.at[pl.ds(j * 128 + 128, 128)]], rows_b, sem_b)
            cpa.wait()
            pltpu.sync_copy(rows_a, odst.at[pl.ds(base + j * 128, 128)])
            cpb.wait()
            pltpu.sync_copy(rows_b, odst.at[pl.ds(base + j * 128 + 128, 128)])
            return 0

        lax.fori_loop(0, T // 256, g_step, 0)

    return k(rank_k, outs)


def _k6_body(x00, x01, x10, x11, f_ref):
    halves = []
    for (xa, xb) in ((x00, x01), (x10, x11)):
        va = xa[...]
        vb = xb[...]
        l0 = va[:, 0:1]
        l1 = vb[:, 0:1]
        m = jnp.maximum(l0, l1)
        e0 = jnp.exp(l0 - m)
        e1 = jnp.exp(l1 - m)
        o = (va * e0 + vb * e1) / (e0 + e1)
        halves.append(o[:, HS:])
    f_ref[0] = jnp.concatenate(halves, axis=1)


def _combine(out_u):
    spec = lambda r: pl.BlockSpec((T, 2 * HS), lambda g, _r=r: (4 * g + _r, 0))
    return pl.pallas_call(
        _k6_body,
        grid=(NH // 2,),
        in_specs=[spec(0), spec(1), spec(2), spec(3)],
        out_specs=pl.BlockSpec((1, T, 2 * HS), lambda g: (0, 0, g)),
        out_shape=jax.ShapeDtypeStruct((1, T, HIDDEN), jnp.float32),
    )(out_u, out_u, out_u, out_u)


def kernel(decoder_states, hidden_states, W_qk, W_v):
    ds = decoder_states[0]
    hs = hidden_states[0]
    wq = W_qk.reshape(HIDDEN, NH, HS).transpose(1, 0, 2)
    wv = W_v.reshape(HIDDEN, NH, HS).transpose(1, 0, 2)
    wkv = jnp.concatenate([wq, wv], axis=-1)
    qtab = _project(ds, wq, _k1q_body)

    rot = jax.random.normal(jax.random.key(HASH_SEED),
                            (NH, HS, NHASH, NB // 2), jnp.float32)
    rot4 = rot.transpose(0, 2, 1, 3)
    rot4 = jnp.concatenate([rot4, jnp.zeros_like(rot4)], axis=2)
    rank_q = _buckets_ranks(qtab, rot4)
    qs = _sc_gather_q(rank_q, qtab)

    kvtab = _project(hs, wkv, _k1kv_body)
    rank_k = _buckets_ranks(kvtab, rot4)
    poskl, kvs = _sc_gather_kv(rank_k, kvtab)
    out_s = _attention(qs, kvs, poskl)
    out_u = _sc_unsort(rank_k, out_s)
    return _combine(out_u)
